# Initial kernel scaffold; baseline (speedup 1.0000x reference)
import functools
import math

import jax
import jax.numpy as jnp
from jax import lax
from jax.experimental import pallas as pl
from jax.experimental.pallas import tpu as pltpu
from jax.experimental.pallas import tpu_sc as plsc

N = 10000
NP = 10240
E = 320000
EMB = 256
DD = 32
CD = 32
HID = 128
H = 8
G = 64
QK = H * HID

NC = 2
NS = 16
NW = NC * NS
EW = E // NW
CH = 40
NCHUNK = EW // CH
ROWS_T = NP // NS
NROW_W = NP // NW

_SC_MESH = dict(core_axis_name="c", subcore_axis_name="s",
                num_cores=NC, num_subcores=NS)

F32 = jnp.float32


def _sds(shape, dtype=F32):
    return jax.ShapeDtypeStruct(shape, dtype)


@functools.partial(
    pl.kernel,
    out_type=(_sds((NP, EMB)), _sds((NP, DD)), _sds((NP, CD))),
    mesh=plsc.VectorSubcoreMesh(**_SC_MESH),
    scratch_types=[
        pltpu.VMEM((64,), jnp.int32),
        pltpu.VMEM((64, EMB), F32),
        pltpu.VMEM((64, DD), F32),
        pltpu.VMEM((64, CD), F32),
        pltpu.SemaphoreType.DMA,
    ],
)
def _embed(x_h, dep_h, chi_h, tn_h, td_h, tc_h, nfa_h, nfb_h, nfc_h,
           ib, nb, db, cb, sem):
    cid = lax.axis_index("c")
    sid = lax.axis_index("s")
    wid = sid * NC + cid
    base0 = wid * NROW_W

    def chunk(i, _):
        base = base0 + i * 64
        pltpu.sync_copy(x_h.at[pl.ds(base, 64)], ib)
        pltpu.async_copy(tn_h.at[ib], nb, sem).wait()
        pltpu.sync_copy(nb, nfa_h.at[pl.ds(base, 64)])
        pltpu.sync_copy(dep_h.at[pl.ds(base, 64)], ib)
        pltpu.async_copy(td_h.at[ib], db, sem).wait()
        pltpu.sync_copy(db, nfb_h.at[pl.ds(base, 64)])
        pltpu.sync_copy(chi_h.at[pl.ds(base, 64)], ib)
        pltpu.async_copy(tc_h.at[ib], cb, sem).wait()
        pltpu.sync_copy(cb, nfc_h.at[pl.ds(base, 64)])
        return _

    lax.fori_loop(0, NROW_W // 64, chunk, None)


@functools.partial(
    pl.kernel,
    out_type=(_sds((E, 16)), _sds((NC, NP, 16))),
    mesh=plsc.VectorSubcoreMesh(**_SC_MESH),
    scratch_types=[
        pltpu.VMEM((NCHUNK, CH), jnp.int32),
        pltpu.VMEM((NCHUNK, CH), jnp.int32),
        pltpu.VMEM((CH, QK), F32),
        pltpu.VMEM((CH, QK), F32),
        pltpu.VMEM((CH, 16), F32),
        pltpu.VMEM((64, 16), F32),
        pltpu.VMEM_SHARED((NP, 16), F32),
        pltpu.SemaphoreType.DMA,
        pltpu.SemaphoreType.DMA,
    ],
)
def _attn_ex(qp_h, kp_h, srcr_h, dstr_h, ex_h, den_h,
             src_i, dst_i, qb, kb, exb, zb, den_sh, sem, sem2):
    cid = lax.axis_index("c")
    sid = lax.axis_index("s")
    wid = sid * NC + cid
    zv = jnp.zeros((16,), F32)

    def zrow(i, _):
        zb[i, :] = zv
        return _

    lax.fori_loop(0, 64, zrow, None)

    def zcopy(t, _):
        pltpu.sync_copy(zb, den_sh.at[pl.ds(sid * ROWS_T + t * 64, 64)])
        return _

    lax.fori_loop(0, ROWS_T // 64, zcopy, None)
    plsc.subcore_barrier()

    pltpu.sync_copy(srcr_h.at[wid], src_i)
    pltpu.sync_copy(dstr_h.at[wid], dst_i)
    scale = 1.0 / math.sqrt(float(HID))
    perm = lax.iota(jnp.int32, 16) ^ 8
    ebase = wid * EW

    def chunk(j, _):
        cq = pltpu.async_copy(qp_h.at[dst_i.at[j]], qb, sem)
        ck = pltpu.async_copy(kp_h.at[src_i.at[j]], kb, sem2)
        cq.wait()
        ck.wait()

        def edge(e, _):
            acc = qb[e, pl.ds(0, 16)] * kb[e, pl.ds(0, 16)]
            for t in range(1, QK // 16):
                acc = acc + qb[e, pl.ds(16 * t, 16)] * kb[e, pl.ds(16 * t, 16)]
            folded = acc + jnp.take(acc, perm,
                                    mode=lax.GatherScatterMode.PROMISE_IN_BOUNDS)
            exb[e, :] = jnp.exp(folded * scale)
            return _

        lax.fori_loop(0, CH, edge, None)
        pltpu.sync_copy(exb, ex_h.at[pl.ds(ebase + j * CH, CH)])
        pltpu.sync_copy(exb, den_sh.at[dst_i.at[j]], add=True)
        return _

    lax.fori_loop(0, NCHUNK, chunk, None)
    plsc.subcore_barrier()

    def dump(t, _):
        off = sid * ROWS_T + t * 64
        pltpu.sync_copy(den_sh.at[pl.ds(off, 64)],
                        den_h.at[cid, pl.ds(off, 64)])
        return _

    lax.fori_loop(0, ROWS_T // 64, dump, None)


@functools.partial(
    pl.kernel,
    out_type=_sds((NC, NP, HID)),
    mesh=plsc.VectorSubcoreMesh(**_SC_MESH),
    scratch_types=[
        pltpu.VMEM((NCHUNK, CH), jnp.int32),
        pltpu.VMEM((NCHUNK, CH), jnp.int32),
        pltpu.VMEM((CH, QK), F32),
        pltpu.VMEM((CH, 16), F32),
        pltpu.VMEM((CH, 16), F32),
        pltpu.VMEM((CH, 16), F32),
        pltpu.VMEM((CH, HID), F32),
        pltpu.VMEM((64, HID), F32),
        pltpu.VMEM_SHARED((NP, HID), F32),
        pltpu.SemaphoreType.DMA,
        pltpu.SemaphoreType.DMA,
        pltpu.SemaphoreType.DMA,
    ],
)
def _attn_agg(v_h, ex_h, den0_h, den1_h, srcr_h, dstr_h, outp_h,
              src_i, dst_i, vb, exb, d0b, d1b, mb, zb, out_sh,
              sem, sem0, sem1):
    cid = lax.axis_index("c")
    sid = lax.axis_index("s")
    wid = sid * NC + cid
    zv = jnp.zeros((16,), F32)

    def zrow(i, _):
        for t in range(HID // 16):
            zb[i, pl.ds(16 * t, 16)] = zv
        return _

    lax.fori_loop(0, 64, zrow, None)

    def zcopy(t, _):
        pltpu.sync_copy(zb, out_sh.at[pl.ds(sid * ROWS_T + t * 64, 64)])
        return _

    lax.fori_loop(0, ROWS_T // 64, zcopy, None)
    plsc.subcore_barrier()

    pltpu.sync_copy(srcr_h.at[wid], src_i)
    pltpu.sync_copy(dstr_h.at[wid], dst_i)
    ebase = wid * EW

    def chunk(j, _):
        cv = pltpu.async_copy(v_h.at[src_i.at[j]], vb, sem)
        c0 = pltpu.async_copy(den0_h.at[dst_i.at[j]], d0b, sem0)
        c1 = pltpu.async_copy(den1_h.at[dst_i.at[j]], d1b, sem1)
        pltpu.sync_copy(ex_h.at[pl.ds(ebase + j * CH, CH)], exb)
        cv.wait()
        c0.wait()
        c1.wait()

        def edge(e, _):
            denv = d0b[e, :] + d1b[e, :]
            alpha = exb[e, :] / jnp.maximum(denv, 1e-16)
            msgs = None
            for hh in range(H):
                ah = jnp.take(alpha, jnp.full((16,), hh, jnp.int32),
                              mode=lax.GatherScatterMode.PROMISE_IN_BOUNDS)
                hb = hh * HID
                if msgs is None:
                    msgs = [ah * vb[e, pl.ds(hb + 16 * t, 16)]
                            for t in range(HID // 16)]
                else:
                    msgs = [msgs[t] + ah * vb[e, pl.ds(hb + 16 * t, 16)]
                            for t in range(HID // 16)]
            for t in range(HID // 16):
                mb[e, pl.ds(16 * t, 16)] = msgs[t]
            return _

        lax.fori_loop(0, CH, edge, None)
        pltpu.sync_copy(mb, out_sh.at[dst_i.at[j]], add=True)
        return _

    lax.fori_loop(0, NCHUNK, chunk, None)
    plsc.subcore_barrier()

    def dump(t, _):
        off = sid * ROWS_T + t * 64
        pltpu.sync_copy(out_sh.at[pl.ds(off, 64)],
                        outp_h.at[cid, pl.ds(off, 64)])
        return _

    lax.fori_loop(0, ROWS_T // 64, dump, None)


RB = 512
NBLK = NP // RB


def _tc_proj0_body(nfa, nfb, nfc, pw, pb, qw, qb, kw, kb, vw, vb, sw, sb,
                   h_o, qp_o, kp_o, v_o, r_o):
    pwv = pw[...]
    h = (jnp.dot(nfa[...], pwv[0:EMB], preferred_element_type=F32)
         + jnp.dot(nfb[...], pwv[EMB:EMB + DD], preferred_element_type=F32)
         + jnp.dot(nfc[...], pwv[EMB + DD:], preferred_element_type=F32)
         + pb[...])
    h = jnp.maximum(h, 0.0)
    h_o[...] = h
    qp_o[...] = jnp.dot(h, qw[...], preferred_element_type=F32) + qb[...]
    kp_o[...] = jnp.dot(h, kw[...], preferred_element_type=F32) + kb[...]
    v_o[...] = jnp.dot(h, vw[...], preferred_element_type=F32) + vb[...]
    r_o[...] = jnp.dot(h, sw[...], preferred_element_type=F32) + sb[...]


def _full(shape):
    nd = len(shape)
    return pl.BlockSpec(shape, lambda i: (0,) * nd)


def _rows(cols):
    return pl.BlockSpec((RB, cols), lambda i: (i, 0))


def _tc_proj0(nfa, nfb, nfc, pw, pb, qw, qb, kw, kb, vw, vb, sw, sb):
    return pl.pallas_call(
        _tc_proj0_body,
        grid=(NBLK,),
        in_specs=[_rows(EMB), _rows(DD), _rows(CD),
                  _full((EMB + DD + CD, EMB)), _full((1, EMB)),
                  _full((EMB, QK)), _full((1, QK)),
                  _full((EMB, QK)), _full((1, QK)),
                  _full((EMB, QK)), _full((1, QK)),
                  _full((EMB, HID)), _full((1, HID))],
        out_specs=[_rows(EMB), _rows(QK), _rows(QK), _rows(QK), _rows(HID)],
        out_shape=[_sds((NP, EMB)), _sds((NP, QK)), _sds((NP, QK)),
                   _sds((NP, QK)), _sds((NP, HID))],
    )(nfa, nfb, nfc, pw, pb, qw, qb, kw, kb, vw, vb, sw, sb)


def _gate(o, rr, bwv):
    z = (jnp.dot(o, bwv[0:HID], preferred_element_type=F32)
         + jnp.dot(rr, bwv[HID:2 * HID], preferred_element_type=F32)
         + jnp.dot(o - rr, bwv[2 * HID:], preferred_element_type=F32))
    g = jax.nn.sigmoid(z)
    return g * rr + (1.0 - g) * o


def _ln_relu(hx, lng, lnb):
    mu = jnp.mean(hx, axis=1, keepdims=True)
    var = jnp.mean((hx - mu) * (hx - mu), axis=1, keepdims=True)
    y = (hx - mu) / jnp.sqrt(var + 1e-5) * lng + lnb
    return jnp.maximum(y, 0.0)


def _tc_mid_body(p0, p1, r, bw, lng, lnb, qw, qb, kw, kb, vw, vb, sw, sb,
                 h0_o, qp_o, kp_o, v_o, r_o):
    o = (p0[...] + p1[...]) * (1.0 / H)
    hx = _gate(o, r[...], bw[...])
    y = _ln_relu(hx, lng[...], lnb[...])
    h0_o[...] = y
    qp_o[...] = jnp.dot(y, qw[...], preferred_element_type=F32) + qb[...]
    kp_o[...] = jnp.dot(y, kw[...], preferred_element_type=F32) + kb[...]
    v_o[...] = jnp.dot(y, vw[...], preferred_element_type=F32) + vb[...]
    r_o[...] = jnp.dot(y, sw[...], preferred_element_type=F32) + sb[...]


def _tc_mid(p0, p1, r, bw, lng, lnb, qw, qb, kw, kb, vw, vb, sw, sb):
    return pl.pallas_call(
        _tc_mid_body,
        grid=(NBLK,),
        in_specs=[_rows(HID), _rows(HID), _rows(HID),
                  _full((3 * HID, 1)), _full((1, HID)), _full((1, HID)),
                  _full((HID, QK)), _full((1, QK)),
                  _full((HID, QK)), _full((1, QK)),
                  _full((HID, QK)), _full((1, QK)),
                  _full((HID, HID)), _full((1, HID))],
        out_specs=[_rows(HID), _rows(QK), _rows(QK), _rows(QK), _rows(HID)],
        out_shape=[_sds((NP, HID)), _sds((NP, QK)), _sds((NP, QK)),
                   _sds((NP, QK)), _sds((NP, HID))],
    )(p0, p1, r, bw, lng, lnb, qw, qb, kw, kb, vw, vb, sw, sb)


def _tc_pool_body(p0, p1, r, h0, bw, lng, lnb, batch3, psum_o, pcnt_o):
    i = pl.program_id(0)
    o = (p0[...] + p1[...]) * (1.0 / H)
    hx = _gate(o, r[...], bw[...]) + h0[...]
    y = _ln_relu(hx, lng[...], lnb[...])
    b = batch3[0]
    seg = lax.broadcasted_iota(jnp.int32, (G, 1), 0)
    oh = (b == seg).astype(F32)

    @pl.when(i == 0)
    def _():
        psum_o[...] = jnp.zeros_like(psum_o)
        pcnt_o[...] = jnp.zeros_like(pcnt_o)

    psum_o[...] += jnp.dot(oh, y, preferred_element_type=F32)
    pcnt_o[...] += jnp.broadcast_to(jnp.sum(oh, axis=1, keepdims=True),
                                    (G, HID))


def _tc_pool(p0, p1, r, h0, bw, lng, lnb, batch3):
    return pl.pallas_call(
        _tc_pool_body,
        grid=(NBLK,),
        in_specs=[_rows(HID), _rows(HID), _rows(HID), _rows(HID),
                  _full((3 * HID, 1)), _full((1, HID)), _full((1, HID)),
                  pl.BlockSpec((1, 1, RB), lambda i: (i, 0, 0))],
        out_specs=[_full((G, HID)), _full((G, HID))],
        out_shape=[_sds((G, HID)), _sds((G, HID))],
    )(p0, p1, r, h0, bw, lng, lnb, batch3)


def _tc_cls_body(psum, pcnt, w1, b1, w2, b2, out_o):
    pooled = psum[...] / jnp.maximum(pcnt[...], 1.0)
    hc = jnp.maximum(jnp.dot(pooled, w1[...], preferred_element_type=F32)
                     + b1[...], 0.0)
    out_o[...] = jnp.dot(hc, w2[...], preferred_element_type=F32) + b2[...]


def _tc_cls(psum, pcnt, w1, b1, w2, b2):
    return pl.pallas_call(
        _tc_cls_body,
        grid=(1,),
        in_specs=[_full((G, HID)), _full((G, HID)),
                  _full((HID, HID // 2)), _full((1, HID // 2)),
                  _full((HID // 2, 1)), _full((1, 1))],
        out_specs=[_full((G, 1))],
        out_shape=[_sds((G, 1))],
    )(psum, pcnt, w1, b1, w2, b2)


def _perm_w(w):
    return w.reshape(-1, H, HID).transpose(0, 2, 1).reshape(-1, H * HID)


def _perm_b(b):
    return b.reshape(H, HID).T.reshape(1, -1)


def kernel(x, edge_index, batch, node_depth, child_index, emb_node, emb_depth,
           emb_child, proj_w, proj_b, l0_qw, l0_qb, l0_kw, l0_kb, l0_vw,
           l0_vb, l0_sw, l0_sb, l0_bw, l0_ln_g, l0_ln_b, l1_qw, l1_qb, l1_kw,
           l1_kb, l1_vw, l1_vb, l1_sw, l1_sb, l1_bw, l1_ln_g, l1_ln_b,
           cls_w1, cls_b1, cls_w2, cls_b2):
    pad = NP - N
    xi = jnp.pad(x.astype(jnp.int32), (0, pad))
    depi = jnp.pad(node_depth.astype(jnp.int32), (0, pad))
    chii = jnp.pad(child_index.astype(jnp.int32), (0, pad))
    batch3 = jnp.pad(batch.astype(jnp.int32), (0, pad),
                     constant_values=G).reshape(NBLK, 1, RB)
    src = edge_index[0].astype(jnp.int32).reshape(NW, NCHUNK, CH)
    dst = edge_index[1].astype(jnp.int32).reshape(NW, NCHUNK, CH)

    nfa, nfb, nfc = _embed(xi, depi, chii, emb_node, emb_depth, emb_child)

    h, qp0, kp0, v0, r0 = _tc_proj0(
        nfa, nfb, nfc, proj_w, proj_b.reshape(1, -1),
        _perm_w(l0_qw), _perm_b(l0_qb), _perm_w(l0_kw), _perm_b(l0_kb),
        l0_vw, l0_vb.reshape(1, -1), l0_sw, l0_sb.reshape(1, -1))

    ex0, den0 = _attn_ex(qp0, kp0, src, dst)
    outp0 = _attn_agg(v0, ex0, den0[0], den0[1], src, dst)

    h0, qp1, kp1, v1, r1 = _tc_mid(
        outp0[0], outp0[1], r0, l0_bw, l0_ln_g.reshape(1, -1),
        l0_ln_b.reshape(1, -1),
        _perm_w(l1_qw), _perm_b(l1_qb), _perm_w(l1_kw), _perm_b(l1_kb),
        l1_vw, l1_vb.reshape(1, -1), l1_sw, l1_sb.reshape(1, -1))

    ex1, den1 = _attn_ex(qp1, kp1, src, dst)
    outp1 = _attn_agg(v1, ex1, den1[0], den1[1], src, dst)

    psum, pcnt = _tc_pool(outp1[0], outp1[1], r1, h0, l1_bw,
                          l1_ln_g.reshape(1, -1), l1_ln_b.reshape(1, -1),
                          batch3)
    return _tc_cls(psum, pcnt, cls_w1, cls_b1.reshape(1, -1),
                   cls_w2, cls_b2.reshape(1, -1))


# speedup vs baseline: 10.8455x; 10.8455x over previous
import functools
import math

import jax
import jax.numpy as jnp
from jax import lax
from jax.experimental import pallas as pl
from jax.experimental.pallas import tpu as pltpu
from jax.experimental.pallas import tpu_sc as plsc

N = 10000
NP = 10240
E = 320000
EMB = 256
DD = 32
CD = 32
HID = 128
H = 8
G = 64
QK = H * HID

NC = 2
NS = 16
NW = NC * NS
EW = E // NW
CH = 16
NCHUNK = EW // CH
IDXB = 25
NREFILL = NCHUNK // IDXB
ROWS_T = NP // NS
NROW_W = NP // NW

_SC_MESH = dict(core_axis_name="c", subcore_axis_name="s",
                num_cores=NC, num_subcores=NS)

F32 = jnp.float32


def _sds(shape, dtype=F32):
    return jax.ShapeDtypeStruct(shape, dtype)


@functools.partial(
    pl.kernel,
    out_type=_sds((NP, EMB)),
    mesh=plsc.VectorSubcoreMesh(**_SC_MESH),
    scratch_types=[
        pltpu.VMEM((64,), jnp.int32),
        pltpu.VMEM((64, EMB), F32),
        pltpu.SemaphoreType.DMA,
    ],
)
def _embed(x_h, tn_h, nfa_h, ib, nb, sem):
    cid = lax.axis_index("c")
    sid = lax.axis_index("s")
    wid = sid * NC + cid
    base0 = wid * NROW_W

    def chunk(i, _):
        base = base0 + i * 64
        pltpu.sync_copy(x_h.at[pl.ds(base, 64)], ib)
        pltpu.async_copy(tn_h.at[ib], nb, sem).wait()
        pltpu.sync_copy(nb, nfa_h.at[pl.ds(base, 64)])
        return _

    lax.fori_loop(0, NROW_W // 64, chunk, None)


@functools.partial(
    pl.kernel,
    out_type=(_sds((E, 16)), _sds((NC, NP, HID))),
    mesh=plsc.VectorSubcoreMesh(**_SC_MESH),
    scratch_types=[
        pltpu.VMEM((IDXB, CH), jnp.int32),
        pltpu.VMEM((IDXB, CH), jnp.int32),
        pltpu.VMEM((CH, QK), F32),
        pltpu.VMEM((CH, QK), F32),
        pltpu.VMEM((CH, 16), F32),
        pltpu.VMEM((CH, HID), F32),
        pltpu.VMEM((16, HID), F32),
        pltpu.VMEM((32,), F32),
        pltpu.VMEM_SHARED((NP, HID), F32),
        pltpu.SemaphoreType.DMA,
        pltpu.SemaphoreType.DMA,
    ],
)
def _attn_ex(qp_h, kp_h, srcr_h, dstr_h, ex_h, den_h,
             src_i, dst_i, qb, kb, exb, exd, zb, tmp, den_sh, sem, sem2):
    cid = lax.axis_index("c")
    sid = lax.axis_index("s")
    wid = sid * NC + cid
    zv = jnp.zeros((16,), F32)

    def zrow(i, _):
        for t in range(HID // 16):
            zb[i, pl.ds(16 * t, 16)] = zv
        return _

    lax.fori_loop(0, 16, zrow, None)

    def zex(e, _):
        for t in range(HID // 16):
            exd[e, pl.ds(16 * t, 16)] = zv
        return _

    lax.fori_loop(0, CH, zex, None)

    def zcopy(t, _):
        pltpu.sync_copy(zb, den_sh.at[pl.ds(sid * ROWS_T + t * 16, 16)])
        return _

    lax.fori_loop(0, ROWS_T // 16, zcopy, None)
    plsc.subcore_barrier()

    scale = 1.0 / math.sqrt(float(HID))
    ebase = wid * EW

    def refill(rj, _):
        pltpu.sync_copy(srcr_h.at[wid, rj], src_i)
        pltpu.sync_copy(dstr_h.at[wid, rj], dst_i)

        def chunk(j, _):
            cq = pltpu.async_copy(qp_h.at[dst_i.at[j]], qb, sem)
            ck = pltpu.async_copy(kp_h.at[src_i.at[j]], kb, sem2)
            cq.wait()
            ck.wait()

            def edge(e, _):
                acc = qb[e, pl.ds(0, 16)] * kb[e, pl.ds(0, 16)]
                for t in range(1, QK // 16):
                    acc = acc + (qb[e, pl.ds(16 * t, 16)]
                                 * kb[e, pl.ds(16 * t, 16)])
                tmp[pl.ds(0, 16)] = acc
                tmp[pl.ds(16, 16)] = acc
                folded = acc + tmp[pl.ds(8, 16)]
                ev = jnp.exp(folded * scale)
                exb[e, :] = ev
                exd[e, pl.ds(0, 16)] = ev
                return _

            lax.fori_loop(0, CH, edge, None)
            pltpu.sync_copy(
                exb, ex_h.at[pl.ds(ebase + (rj * IDXB + j) * CH, CH)])
            pltpu.sync_copy(exd, den_sh.at[dst_i.at[j]], add=True)
            return _

        lax.fori_loop(0, IDXB, chunk, None)
        return _

    lax.fori_loop(0, NREFILL, refill, None)
    plsc.subcore_barrier()

    def dump(t, _):
        off = sid * ROWS_T + t * 64
        pltpu.sync_copy(den_sh.at[pl.ds(off, 64)],
                        den_h.at[cid, pl.ds(off, 64)])
        return _

    lax.fori_loop(0, ROWS_T // 64, dump, None)


@functools.partial(
    pl.kernel,
    out_type=_sds((NC, NP, HID)),
    mesh=plsc.VectorSubcoreMesh(**_SC_MESH),
    scratch_types=[
        pltpu.VMEM((IDXB, CH), jnp.int32),
        pltpu.VMEM((IDXB, CH), jnp.int32),
        pltpu.VMEM((CH, QK), F32),
        pltpu.VMEM((CH, 16), F32),
        pltpu.VMEM((CH, HID), F32),
        pltpu.VMEM((CH, HID), F32),
        pltpu.VMEM((CH, HID), F32),
        pltpu.VMEM((16, HID), F32),
        pltpu.VMEM_SHARED((NP, HID), F32),
        pltpu.SemaphoreType.DMA,
        pltpu.SemaphoreType.DMA,
        pltpu.SemaphoreType.DMA,
    ],
)
def _attn_agg(v_h, ex_h, den0_h, den1_h, srcr_h, dstr_h, outp_h,
              src_i, dst_i, vb, exb, d0b, d1b, mb, zb, out_sh,
              sem, sem0, sem1):
    cid = lax.axis_index("c")
    sid = lax.axis_index("s")
    wid = sid * NC + cid
    zv = jnp.zeros((16,), F32)

    def zrow(i, _):
        for t in range(HID // 16):
            zb[i, pl.ds(16 * t, 16)] = zv
        return _

    lax.fori_loop(0, 16, zrow, None)

    def zcopy(t, _):
        pltpu.sync_copy(zb, out_sh.at[pl.ds(sid * ROWS_T + t * 16, 16)])
        return _

    lax.fori_loop(0, ROWS_T // 16, zcopy, None)
    plsc.subcore_barrier()

    ebase = wid * EW

    def refill(rj, _):
        pltpu.sync_copy(srcr_h.at[wid, rj], src_i)
        pltpu.sync_copy(dstr_h.at[wid, rj], dst_i)

        def chunk(j, _):
            cv = pltpu.async_copy(v_h.at[src_i.at[j]], vb, sem)
            c0 = pltpu.async_copy(den0_h.at[dst_i.at[j]], d0b, sem0)
            c1 = pltpu.async_copy(den1_h.at[dst_i.at[j]], d1b, sem1)
            pltpu.sync_copy(
                ex_h.at[pl.ds(ebase + (rj * IDXB + j) * CH, CH)], exb)
            cv.wait()
            c0.wait()
            c1.wait()

            def edge(e, _):
                denv = d0b[e, pl.ds(0, 16)] + d1b[e, pl.ds(0, 16)]
                alpha = exb[e, :] / jnp.maximum(denv, 1e-16)
                msgs = None
                for hh in range(H):
                    ah = alpha[hh]
                    hb = hh * HID
                    if msgs is None:
                        msgs = [ah * vb[e, pl.ds(hb + 16 * t, 16)]
                                for t in range(HID // 16)]
                    else:
                        msgs = [msgs[t] + ah * vb[e, pl.ds(hb + 16 * t, 16)]
                                for t in range(HID // 16)]
                for t in range(HID // 16):
                    mb[e, pl.ds(16 * t, 16)] = msgs[t]
                return _

            lax.fori_loop(0, CH, edge, None)
            pltpu.sync_copy(mb, out_sh.at[dst_i.at[j]], add=True)
            return _

        lax.fori_loop(0, IDXB, chunk, None)
        return _

    lax.fori_loop(0, NREFILL, refill, None)
    plsc.subcore_barrier()

    def dump(t, _):
        off = sid * ROWS_T + t * 64
        pltpu.sync_copy(out_sh.at[pl.ds(off, 64)],
                        outp_h.at[cid, pl.ds(off, 64)])
        return _

    lax.fori_loop(0, ROWS_T // 64, dump, None)


RB = 512
NBLK = NP // RB


DDEP = 51
DCHI = 21


def _onehot_lookup(ids3, table, n_ids):
    ids = ids3[0]
    oh = (lax.broadcasted_iota(jnp.int32, (n_ids, 1), 0) == ids).astype(F32)
    return lax.dot_general(oh, table, (((0,), (0,)), ((), ())),
                           preferred_element_type=F32)


def _tc_proj0_body(nfa, dep3, chi3, td, tc, pw, pb, qw, qb, kw, kb, vw, vb,
                   sw, sb, h_o, qp_o, kp_o, v_o, r_o):
    pwv = pw[...]
    nfb = _onehot_lookup(dep3, td[...], DDEP)
    nfc = _onehot_lookup(chi3, tc[...], DCHI)
    h = (jnp.dot(nfa[...], pwv[0:EMB], preferred_element_type=F32)
         + jnp.dot(nfb, pwv[EMB:EMB + DD], preferred_element_type=F32)
         + jnp.dot(nfc, pwv[EMB + DD:], preferred_element_type=F32)
         + pb[...])
    h = jnp.maximum(h, 0.0)
    h_o[...] = h
    qp_o[...] = jnp.dot(h, qw[...], preferred_element_type=F32) + qb[...]
    kp_o[...] = jnp.dot(h, kw[...], preferred_element_type=F32) + kb[...]
    v_o[...] = jnp.dot(h, vw[...], preferred_element_type=F32) + vb[...]
    r_o[...] = jnp.dot(h, sw[...], preferred_element_type=F32) + sb[...]


def _full(shape):
    nd = len(shape)
    return pl.BlockSpec(shape, lambda i: (0,) * nd)


def _rows(cols):
    return pl.BlockSpec((RB, cols), lambda i: (i, 0))


def _tc_proj0(nfa, dep3, chi3, td, tc, pw, pb, qw, qb, kw, kb, vw, vb,
              sw, sb):
    return pl.pallas_call(
        _tc_proj0_body,
        grid=(NBLK,),
        in_specs=[_rows(EMB),
                  pl.BlockSpec((1, 1, RB), lambda i: (i, 0, 0)),
                  pl.BlockSpec((1, 1, RB), lambda i: (i, 0, 0)),
                  _full((DDEP, DD)), _full((DCHI, CD)),
                  _full((EMB + DD + CD, EMB)), _full((1, EMB)),
                  _full((EMB, QK)), _full((1, QK)),
                  _full((EMB, QK)), _full((1, QK)),
                  _full((EMB, QK)), _full((1, QK)),
                  _full((EMB, HID)), _full((1, HID))],
        out_specs=[_rows(EMB), _rows(QK), _rows(QK), _rows(QK), _rows(HID)],
        out_shape=[_sds((NP, EMB)), _sds((NP, QK)), _sds((NP, QK)),
                   _sds((NP, QK)), _sds((NP, HID))],
    )(nfa, dep3, chi3, td, tc, pw, pb, qw, qb, kw, kb, vw, vb, sw, sb)


def _gate(o, rr, bwv):
    z = (jnp.dot(o, bwv[0:HID], preferred_element_type=F32)
         + jnp.dot(rr, bwv[HID:2 * HID], preferred_element_type=F32)
         + jnp.dot(o - rr, bwv[2 * HID:], preferred_element_type=F32))
    g = jax.nn.sigmoid(z)
    return g * rr + (1.0 - g) * o


def _ln_relu(hx, lng, lnb):
    mu = jnp.mean(hx, axis=1, keepdims=True)
    var = jnp.mean((hx - mu) * (hx - mu), axis=1, keepdims=True)
    y = (hx - mu) / jnp.sqrt(var + 1e-5) * lng + lnb
    return jnp.maximum(y, 0.0)


def _tc_mid_body(p0, p1, r, bw, lng, lnb, qw, qb, kw, kb, vw, vb, sw, sb,
                 h0_o, qp_o, kp_o, v_o, r_o):
    o = (p0[...] + p1[...]) * (1.0 / H)
    hx = _gate(o, r[...], bw[...])
    y = _ln_relu(hx, lng[...], lnb[...])
    h0_o[...] = y
    qp_o[...] = jnp.dot(y, qw[...], preferred_element_type=F32) + qb[...]
    kp_o[...] = jnp.dot(y, kw[...], preferred_element_type=F32) + kb[...]
    v_o[...] = jnp.dot(y, vw[...], preferred_element_type=F32) + vb[...]
    r_o[...] = jnp.dot(y, sw[...], preferred_element_type=F32) + sb[...]


def _tc_mid(p0, p1, r, bw, lng, lnb, qw, qb, kw, kb, vw, vb, sw, sb):
    return pl.pallas_call(
        _tc_mid_body,
        grid=(NBLK,),
        in_specs=[_rows(HID), _rows(HID), _rows(HID),
                  _full((3 * HID, 1)), _full((1, HID)), _full((1, HID)),
                  _full((HID, QK)), _full((1, QK)),
                  _full((HID, QK)), _full((1, QK)),
                  _full((HID, QK)), _full((1, QK)),
                  _full((HID, HID)), _full((1, HID))],
        out_specs=[_rows(HID), _rows(QK), _rows(QK), _rows(QK), _rows(HID)],
        out_shape=[_sds((NP, HID)), _sds((NP, QK)), _sds((NP, QK)),
                   _sds((NP, QK)), _sds((NP, HID))],
    )(p0, p1, r, bw, lng, lnb, qw, qb, kw, kb, vw, vb, sw, sb)


def _tc_pool_body(p0, p1, r, h0, bw, lng, lnb, batch3, psum_o, pcnt_o):
    i = pl.program_id(0)
    o = (p0[...] + p1[...]) * (1.0 / H)
    hx = _gate(o, r[...], bw[...]) + h0[...]
    y = _ln_relu(hx, lng[...], lnb[...])
    b = batch3[0]
    seg = lax.broadcasted_iota(jnp.int32, (G, 1), 0)
    oh = (b == seg).astype(F32)

    @pl.when(i == 0)
    def _():
        psum_o[...] = jnp.zeros_like(psum_o)
        pcnt_o[...] = jnp.zeros_like(pcnt_o)

    psum_o[...] += jnp.dot(oh, y, preferred_element_type=F32)
    pcnt_o[...] += jnp.broadcast_to(jnp.sum(oh, axis=1, keepdims=True),
                                    (G, HID))


def _tc_pool(p0, p1, r, h0, bw, lng, lnb, batch3):
    return pl.pallas_call(
        _tc_pool_body,
        grid=(NBLK,),
        in_specs=[_rows(HID), _rows(HID), _rows(HID), _rows(HID),
                  _full((3 * HID, 1)), _full((1, HID)), _full((1, HID)),
                  pl.BlockSpec((1, 1, RB), lambda i: (i, 0, 0))],
        out_specs=[_full((G, HID)), _full((G, HID))],
        out_shape=[_sds((G, HID)), _sds((G, HID))],
    )(p0, p1, r, h0, bw, lng, lnb, batch3)


def _tc_cls_body(psum, pcnt, w1, b1, w2, b2, out_o):
    pooled = psum[...] / jnp.maximum(pcnt[...], 1.0)
    hc = jnp.maximum(jnp.dot(pooled, w1[...], preferred_element_type=F32)
                     + b1[...], 0.0)
    out_o[...] = jnp.dot(hc, w2[...], preferred_element_type=F32) + b2[...]


def _tc_cls(psum, pcnt, w1, b1, w2, b2):
    return pl.pallas_call(
        _tc_cls_body,
        grid=(1,),
        in_specs=[_full((G, HID)), _full((G, HID)),
                  _full((HID, HID // 2)), _full((1, HID // 2)),
                  _full((HID // 2, 1)), _full((1, 1))],
        out_specs=[_full((G, 1))],
        out_shape=[_sds((G, 1))],
    )(psum, pcnt, w1, b1, w2, b2)


def _perm_w(w):
    return w.reshape(-1, H, HID).transpose(0, 2, 1).reshape(-1, H * HID)


def _perm_b(b):
    return b.reshape(H, HID).T.reshape(1, -1)


def kernel(x, edge_index, batch, node_depth, child_index, emb_node, emb_depth,
           emb_child, proj_w, proj_b, l0_qw, l0_qb, l0_kw, l0_kb, l0_vw,
           l0_vb, l0_sw, l0_sb, l0_bw, l0_ln_g, l0_ln_b, l1_qw, l1_qb, l1_kw,
           l1_kb, l1_vw, l1_vb, l1_sw, l1_sb, l1_bw, l1_ln_g, l1_ln_b,
           cls_w1, cls_b1, cls_w2, cls_b2):
    pad = NP - N
    xi = jnp.pad(x.astype(jnp.int32), (0, pad))
    dep3 = jnp.pad(node_depth.astype(jnp.int32), (0, pad)).reshape(
        NBLK, 1, RB)
    chi3 = jnp.pad(child_index.astype(jnp.int32), (0, pad)).reshape(
        NBLK, 1, RB)
    batch3 = jnp.pad(batch.astype(jnp.int32), (0, pad),
                     constant_values=G).reshape(NBLK, 1, RB)
    src = edge_index[0].astype(jnp.int32).reshape(NW, NREFILL, IDXB, CH)
    dst = edge_index[1].astype(jnp.int32).reshape(NW, NREFILL, IDXB, CH)

    nfa = _embed(xi, emb_node)

    h, qp0, kp0, v0, r0 = _tc_proj0(
        nfa, dep3, chi3, emb_depth, emb_child,
        proj_w, proj_b.reshape(1, -1),
        _perm_w(l0_qw), _perm_b(l0_qb), _perm_w(l0_kw), _perm_b(l0_kb),
        l0_vw, l0_vb.reshape(1, -1), l0_sw, l0_sb.reshape(1, -1))

    ex0, den0 = _attn_ex(qp0, kp0, src, dst)
    outp0 = _attn_agg(v0, ex0, den0[0], den0[1], src, dst)

    h0, qp1, kp1, v1, r1 = _tc_mid(
        outp0[0], outp0[1], r0, l0_bw, l0_ln_g.reshape(1, -1),
        l0_ln_b.reshape(1, -1),
        _perm_w(l1_qw), _perm_b(l1_qb), _perm_w(l1_kw), _perm_b(l1_kb),
        l1_vw, l1_vb.reshape(1, -1), l1_sw, l1_sb.reshape(1, -1))

    ex1, den1 = _attn_ex(qp1, kp1, src, dst)
    outp1 = _attn_agg(v1, ex1, den1[0], den1[1], src, dst)

    psum, pcnt = _tc_pool(outp1[0], outp1[1], r1, h0, l1_bw,
                          l1_ln_g.reshape(1, -1), l1_ln_b.reshape(1, -1),
                          batch3)
    return _tc_cls(psum, pcnt, cls_w1, cls_b1.reshape(1, -1),
                   cls_w2, cls_b2.reshape(1, -1))[0]


# speedup vs baseline: 13.1195x; 1.2097x over previous
import functools
import math

import jax
import jax.numpy as jnp
from jax import lax
from jax.experimental import pallas as pl
from jax.experimental.pallas import tpu as pltpu
from jax.experimental.pallas import tpu_sc as plsc

N = 10000
NP = 10240
E = 320000
EMB = 256
DD = 32
CD = 32
HID = 128
H = 8
G = 64
QK = H * HID
QKP = QK // 2

NC = 2
NS = 16
NW = NC * NS
EW = E // NW
CH = 16
NCHUNK = EW // CH
IDXB = 25
NREFILL = NCHUNK // IDXB
AGG_IDXB = 5
AGG_NREFILL = NCHUNK // AGG_IDXB
ROWS_T = NP // NS
NROW_W = NP // NW

LANES = [0, 4, 1, 5, 2, 6, 3, 7]

_SC_MESH = dict(core_axis_name="c", subcore_axis_name="s",
                num_cores=NC, num_subcores=NS)

F32 = jnp.float32
BF16 = jnp.bfloat16


def _sds(shape, dtype=F32):
    return jax.ShapeDtypeStruct(shape, dtype)


@functools.partial(
    pl.kernel,
    out_type=_sds((NP, EMB)),
    mesh=plsc.VectorSubcoreMesh(**_SC_MESH),
    scratch_types=[
        pltpu.VMEM((64,), jnp.int32),
        pltpu.VMEM((64, EMB), F32),
        pltpu.SemaphoreType.DMA,
    ],
)
def _embed(x_h, tn_h, nfa_h, ib, nb, sem):
    cid = lax.axis_index("c")
    sid = lax.axis_index("s")
    wid = sid * NC + cid
    base0 = wid * NROW_W

    def chunk(i, _):
        base = base0 + i * 64
        pltpu.sync_copy(x_h.at[pl.ds(base, 64)], ib)
        pltpu.async_copy(tn_h.at[ib], nb, sem).wait()
        pltpu.sync_copy(nb, nfa_h.at[pl.ds(base, 64)])
        return _

    lax.fori_loop(0, NROW_W // 64, chunk, None)


@functools.partial(
    pl.kernel,
    out_type=(_sds((E, 16)), _sds((NC, NP, HID))),
    mesh=plsc.VectorSubcoreMesh(**_SC_MESH),
    scratch_types=[
        pltpu.VMEM((IDXB, CH), jnp.int32),
        pltpu.VMEM((IDXB, CH), jnp.int32),
        pltpu.VMEM((CH, QKP), F32),
        pltpu.VMEM((CH, QKP), F32),
        pltpu.VMEM((CH, QKP), F32),
        pltpu.VMEM((CH, QKP), F32),
        pltpu.VMEM((CH, 16), F32),
        pltpu.VMEM((CH, HID), F32),
        pltpu.VMEM((16, HID), F32),
        pltpu.VMEM((32,), F32),
        pltpu.VMEM_SHARED((NP, HID), F32),
        pltpu.SemaphoreType.DMA,
        pltpu.SemaphoreType.DMA,
        pltpu.SemaphoreType.DMA,
        pltpu.SemaphoreType.DMA,
    ],
)
def _attn_ex(qp_h, kp_h, srcr_h, dstr_h, ex_h, den_h,
             src_i, dst_i, qba, kba, qbb, kbb, exb, exd, zb, tmp, den_sh,
             sqa, ska, sqb, skb):
    cid = lax.axis_index("c")
    sid = lax.axis_index("s")
    wid = sid * NC + cid
    zv = jnp.zeros((16,), F32)

    def zrow(i, _):
        for t in range(HID // 16):
            zb[i, pl.ds(16 * t, 16)] = zv
        return _

    lax.fori_loop(0, 16, zrow, None)

    def zex(e, _):
        for t in range(HID // 16):
            exd[e, pl.ds(16 * t, 16)] = zv
        return _

    lax.fori_loop(0, CH, zex, None)

    def zcopy(t, _):
        pltpu.sync_copy(zb, den_sh.at[pl.ds(sid * ROWS_T + t * 16, 16)])
        return _

    lax.fori_loop(0, ROWS_T // 16, zcopy, None)
    plsc.subcore_barrier()

    scale = 1.0 / math.sqrt(float(HID))
    ebase = wid * EW
    mask_hi = jnp.int32(-65536)

    def split(v16):
        wi = lax.bitcast_convert_type(v16, jnp.int32)
        ev = lax.bitcast_convert_type(lax.shift_left(wi, 16), F32)
        od = lax.bitcast_convert_type(lax.bitwise_and(wi, mask_hi), F32)
        return ev, od

    def start(j, qb, kb, sq, sk):
        pltpu.async_copy(qp_h.at[dst_i.at[j]], qb, sq)
        pltpu.async_copy(kp_h.at[src_i.at[j]], kb, sk)

    def wait(qb, kb, sq, sk):
        pltpu.make_async_copy(qp_h.at[dst_i.at[0]], qb, sq).wait()
        pltpu.make_async_copy(kp_h.at[src_i.at[0]], kb, sk).wait()

    def compute(rj, j, qb, kb):
        def edge(e, _):
            acc_e = None
            acc_o = None
            for t in range(QKP // 16):
                qe, qo = split(qb[e, pl.ds(16 * t, 16)])
                ke, ko = split(kb[e, pl.ds(16 * t, 16)])
                if acc_e is None:
                    acc_e = qe * ke
                    acc_o = qo * ko
                else:
                    acc_e = acc_e + qe * ke
                    acc_o = acc_o + qo * ko
            folded = []
            for a in (acc_e, acc_o):
                tmp[pl.ds(0, 16)] = a
                tmp[pl.ds(16, 16)] = a
                s1 = a + tmp[pl.ds(8, 16)]
                tmp[pl.ds(0, 16)] = s1
                tmp[pl.ds(16, 16)] = s1
                folded.append(s1 + tmp[pl.ds(4, 16)])
            tmp[pl.ds(0, 16)] = folded[0]
            tmp[pl.ds(4, 16)] = folded[1]
            ev = jnp.exp(tmp[pl.ds(0, 16)] * scale)
            exb[e, :] = ev
            exd[e, pl.ds(0, 16)] = ev
            return _

        lax.fori_loop(0, CH, edge, None)
        pltpu.sync_copy(exb,
                        ex_h.at[pl.ds(ebase + (rj * IDXB + j) * CH, CH)])
        pltpu.sync_copy(exd, den_sh.at[dst_i.at[j]], add=True)

    def refill(rj, _):
        pltpu.sync_copy(srcr_h.at[wid, rj], src_i)
        pltpu.sync_copy(dstr_h.at[wid, rj], dst_i)
        start(0, qba, kba, sqa, ska)

        def pair(jj, _):
            j0 = 2 * jj
            start(j0 + 1, qbb, kbb, sqb, skb)
            wait(qba, kba, sqa, ska)
            compute(rj, j0, qba, kba)
            start(j0 + 2, qba, kba, sqa, ska)
            wait(qbb, kbb, sqb, skb)
            compute(rj, j0 + 1, qbb, kbb)
            return _

        lax.fori_loop(0, (IDXB - 1) // 2, pair, None)
        wait(qba, kba, sqa, ska)
        compute(rj, IDXB - 1, qba, kba)
        return _

    lax.fori_loop(0, NREFILL, refill, None)
    plsc.subcore_barrier()

    def dump(t, _):
        off = sid * ROWS_T + t * 64
        pltpu.sync_copy(den_sh.at[pl.ds(off, 64)],
                        den_h.at[cid, pl.ds(off, 64)])
        return _

    lax.fori_loop(0, ROWS_T // 64, dump, None)


@functools.partial(
    pl.kernel,
    out_type=_sds((NC, NP, HID)),
    mesh=plsc.VectorSubcoreMesh(**_SC_MESH),
    scratch_types=[
        pltpu.VMEM((AGG_IDXB, CH), jnp.int32),
        pltpu.VMEM((AGG_IDXB, CH), jnp.int32),
        pltpu.VMEM((CH, QK), F32),
        pltpu.VMEM((CH, QK), F32),
        pltpu.VMEM((CH, 16), F32),
        pltpu.VMEM((CH, HID), F32),
        pltpu.VMEM((CH, HID), F32),
        pltpu.VMEM((CH, HID), F32),
        pltpu.VMEM((8, HID), F32),
        pltpu.VMEM_SHARED((NP, HID), F32),
        pltpu.SemaphoreType.DMA,
        pltpu.SemaphoreType.DMA,
        pltpu.SemaphoreType.DMA,
        pltpu.SemaphoreType.DMA,
    ],
)
def _attn_agg(v_h, ex_h, den0_h, den1_h, srcr_h, dstr_h, outp_h,
              src_i, dst_i, vba, vbb, exb, d0b, d1b, mb, zb, out_sh,
              sva, svb, sd0, sd1):
    cid = lax.axis_index("c")
    sid = lax.axis_index("s")
    wid = sid * NC + cid
    zv = jnp.zeros((16,), F32)

    def zrow(i, _):
        for t in range(HID // 16):
            zb[i, pl.ds(16 * t, 16)] = zv
        return _

    lax.fori_loop(0, 8, zrow, None)

    def zcopy(t, _):
        pltpu.sync_copy(zb, out_sh.at[pl.ds(sid * ROWS_T + t * 8, 8)])
        return _

    lax.fori_loop(0, ROWS_T // 8, zcopy, None)
    plsc.subcore_barrier()

    ebase = wid * EW

    def start(j, vb, sv):
        pltpu.async_copy(v_h.at[src_i.at[j]], vb, sv)

    def wait(vb, sv):
        pltpu.make_async_copy(v_h.at[src_i.at[0]], vb, sv).wait()

    def compute(rj, j, vb):
        c0 = pltpu.async_copy(den0_h.at[dst_i.at[j]], d0b, sd0)
        c1 = pltpu.async_copy(den1_h.at[dst_i.at[j]], d1b, sd1)
        pltpu.sync_copy(
            ex_h.at[pl.ds(ebase + (rj * AGG_IDXB + j) * CH, CH)], exb)
        c0.wait()
        c1.wait()

        def edge(e, _):
            denv = d0b[e, pl.ds(0, 16)] + d1b[e, pl.ds(0, 16)]
            alpha = exb[e, :] / jnp.maximum(denv, 1e-16)
            msgs = None
            for hh in range(H):
                ah = alpha[LANES[hh]]
                hb = hh * HID
                if msgs is None:
                    msgs = [ah * vb[e, pl.ds(hb + 16 * t, 16)]
                            for t in range(HID // 16)]
                else:
                    msgs = [msgs[t] + ah * vb[e, pl.ds(hb + 16 * t, 16)]
                            for t in range(HID // 16)]
            for t in range(HID // 16):
                mb[e, pl.ds(16 * t, 16)] = msgs[t]
            return _

        lax.fori_loop(0, CH, edge, None)
        pltpu.sync_copy(mb, out_sh.at[dst_i.at[j]], add=True)

    def refill(rj, _):
        pltpu.sync_copy(srcr_h.at[wid, rj], src_i)
        pltpu.sync_copy(dstr_h.at[wid, rj], dst_i)
        start(0, vba, sva)

        def pair(jj, _):
            j0 = 2 * jj
            start(j0 + 1, vbb, svb)
            wait(vba, sva)
            compute(rj, j0, vba)
            start(j0 + 2, vba, sva)
            wait(vbb, svb)
            compute(rj, j0 + 1, vbb)
            return _

        lax.fori_loop(0, (AGG_IDXB - 1) // 2, pair, None)
        wait(vba, sva)
        compute(rj, AGG_IDXB - 1, vba)
        return _

    lax.fori_loop(0, AGG_NREFILL, refill, None)
    plsc.subcore_barrier()

    def dump(t, _):
        off = sid * ROWS_T + t * 64
        pltpu.sync_copy(out_sh.at[pl.ds(off, 64)],
                        outp_h.at[cid, pl.ds(off, 64)])
        return _

    lax.fori_loop(0, ROWS_T // 64, dump, None)


RB = 512
NBLK = NP // RB

DDEP = 51
DCHI = 21


def _onehot_lookup(ids3, table, n_ids):
    ids = ids3[0]
    oh = (lax.broadcasted_iota(jnp.int32, (n_ids, 1), 0) == ids).astype(F32)
    return lax.dot_general(oh, table, (((0,), (0,)), ((), ())),
                           preferred_element_type=F32)


def _tc_proj0_body(nfa, dep3, chi3, td, tc, pw, pb, qw, qb, kw, kb, vw, vb,
                   sw, sb, h_o, qp_o, kp_o, v_o, r_o):
    pwv = pw[...]
    nfb = _onehot_lookup(dep3, td[...], DDEP)
    nfc = _onehot_lookup(chi3, tc[...], DCHI)
    h = (jnp.dot(nfa[...], pwv[0:EMB], preferred_element_type=F32)
         + jnp.dot(nfb, pwv[EMB:EMB + DD], preferred_element_type=F32)
         + jnp.dot(nfc, pwv[EMB + DD:], preferred_element_type=F32)
         + pb[...])
    h = jnp.maximum(h, 0.0)
    h_o[...] = h
    qp_o[...] = (jnp.dot(h, qw[...], preferred_element_type=F32)
                 + qb[...]).astype(BF16)
    kp_o[...] = (jnp.dot(h, kw[...], preferred_element_type=F32)
                 + kb[...]).astype(BF16)
    v_o[...] = jnp.dot(h, vw[...], preferred_element_type=F32) + vb[...]
    r_o[...] = jnp.dot(h, sw[...], preferred_element_type=F32) + sb[...]


def _pack_words(a):
    return lax.bitcast_convert_type(a.reshape(NP, QKP, 2), F32)


def _full(shape):
    nd = len(shape)
    return pl.BlockSpec(shape, lambda i: (0,) * nd)


def _rows(cols):
    return pl.BlockSpec((RB, cols), lambda i: (i, 0))


def _tc_proj0(nfa, dep3, chi3, td, tc, pw, pb, qw, qb, kw, kb, vw, vb,
              sw, sb):
    return pl.pallas_call(
        _tc_proj0_body,
        grid=(NBLK,),
        in_specs=[_rows(EMB),
                  pl.BlockSpec((1, 1, RB), lambda i: (i, 0, 0)),
                  pl.BlockSpec((1, 1, RB), lambda i: (i, 0, 0)),
                  _full((DDEP, DD)), _full((DCHI, CD)),
                  _full((EMB + DD + CD, EMB)), _full((1, EMB)),
                  _full((EMB, QK)), _full((1, QK)),
                  _full((EMB, QK)), _full((1, QK)),
                  _full((EMB, QK)), _full((1, QK)),
                  _full((EMB, HID)), _full((1, HID))],
        out_specs=[_rows(EMB), _rows(QK), _rows(QK), _rows(QK),
                   _rows(HID)],
        out_shape=[_sds((NP, EMB)), _sds((NP, QK), BF16),
                   _sds((NP, QK), BF16), _sds((NP, QK)), _sds((NP, HID))],
    )(nfa, dep3, chi3, td, tc, pw, pb, qw, qb, kw, kb, vw, vb, sw, sb)


def _gate(o, rr, bwv):
    z = (jnp.dot(o, bwv[0:HID], preferred_element_type=F32)
         + jnp.dot(rr, bwv[HID:2 * HID], preferred_element_type=F32)
         + jnp.dot(o - rr, bwv[2 * HID:], preferred_element_type=F32))
    g = jax.nn.sigmoid(z)
    return g * rr + (1.0 - g) * o


def _ln_relu(hx, lng, lnb):
    mu = jnp.mean(hx, axis=1, keepdims=True)
    var = jnp.mean((hx - mu) * (hx - mu), axis=1, keepdims=True)
    y = (hx - mu) / jnp.sqrt(var + 1e-5) * lng + lnb
    return jnp.maximum(y, 0.0)


def _tc_mid_body(p0, p1, r, bw, lng, lnb, qw, qb, kw, kb, vw, vb, sw, sb,
                 h0_o, qp_o, kp_o, v_o, r_o):
    o = (p0[...] + p1[...]) * (1.0 / H)
    hx = _gate(o, r[...], bw[...])
    y = _ln_relu(hx, lng[...], lnb[...])
    h0_o[...] = y
    qp_o[...] = (jnp.dot(y, qw[...], preferred_element_type=F32)
                 + qb[...]).astype(BF16)
    kp_o[...] = (jnp.dot(y, kw[...], preferred_element_type=F32)
                 + kb[...]).astype(BF16)
    v_o[...] = jnp.dot(y, vw[...], preferred_element_type=F32) + vb[...]
    r_o[...] = jnp.dot(y, sw[...], preferred_element_type=F32) + sb[...]


def _tc_mid(p0, p1, r, bw, lng, lnb, qw, qb, kw, kb, vw, vb, sw, sb):
    return pl.pallas_call(
        _tc_mid_body,
        grid=(NBLK,),
        in_specs=[_rows(HID), _rows(HID), _rows(HID),
                  _full((3 * HID, 1)), _full((1, HID)), _full((1, HID)),
                  _full((HID, QK)), _full((1, QK)),
                  _full((HID, QK)), _full((1, QK)),
                  _full((HID, QK)), _full((1, QK)),
                  _full((HID, HID)), _full((1, HID))],
        out_specs=[_rows(HID), _rows(QK), _rows(QK), _rows(QK),
                   _rows(HID)],
        out_shape=[_sds((NP, HID)), _sds((NP, QK), BF16),
                   _sds((NP, QK), BF16), _sds((NP, QK)), _sds((NP, HID))],
    )(p0, p1, r, bw, lng, lnb, qw, qb, kw, kb, vw, vb, sw, sb)


def _tc_pool_body(p0, p1, r, h0, bw, lng, lnb, batch3, psum_o, pcnt_o):
    i = pl.program_id(0)
    o = (p0[...] + p1[...]) * (1.0 / H)
    hx = _gate(o, r[...], bw[...]) + h0[...]
    y = _ln_relu(hx, lng[...], lnb[...])
    b = batch3[0]
    seg = lax.broadcasted_iota(jnp.int32, (G, 1), 0)
    oh = (b == seg).astype(F32)

    @pl.when(i == 0)
    def _():
        psum_o[...] = jnp.zeros_like(psum_o)
        pcnt_o[...] = jnp.zeros_like(pcnt_o)

    psum_o[...] += jnp.dot(oh, y, preferred_element_type=F32)
    pcnt_o[...] += jnp.broadcast_to(jnp.sum(oh, axis=1, keepdims=True),
                                    (G, HID))


def _tc_pool(p0, p1, r, h0, bw, lng, lnb, batch3):
    return pl.pallas_call(
        _tc_pool_body,
        grid=(NBLK,),
        in_specs=[_rows(HID), _rows(HID), _rows(HID), _rows(HID),
                  _full((3 * HID, 1)), _full((1, HID)), _full((1, HID)),
                  pl.BlockSpec((1, 1, RB), lambda i: (i, 0, 0))],
        out_specs=[_full((G, HID)), _full((G, HID))],
        out_shape=[_sds((G, HID)), _sds((G, HID))],
    )(p0, p1, r, h0, bw, lng, lnb, batch3)


def _tc_cls_body(psum, pcnt, w1, b1, w2, b2, out_o):
    pooled = psum[...] / jnp.maximum(pcnt[...], 1.0)
    hc = jnp.maximum(jnp.dot(pooled, w1[...], preferred_element_type=F32)
                     + b1[...], 0.0)
    out_o[...] = jnp.dot(hc, w2[...], preferred_element_type=F32) + b2[...]


def _tc_cls(psum, pcnt, w1, b1, w2, b2):
    return pl.pallas_call(
        _tc_cls_body,
        grid=(1,),
        in_specs=[_full((G, HID)), _full((G, HID)),
                  _full((HID, HID // 2)), _full((1, HID // 2)),
                  _full((HID // 2, 1)), _full((1, 1))],
        out_specs=[_full((G, 1))],
        out_shape=[_sds((G, 1))],
    )(psum, pcnt, w1, b1, w2, b2)


def _perm_w(w):
    return w.reshape(-1, H, HID).transpose(0, 2, 1).reshape(-1, H * HID)


def _perm_b(b):
    return b.reshape(H, HID).T.reshape(1, -1)


def kernel(x, edge_index, batch, node_depth, child_index, emb_node, emb_depth,
           emb_child, proj_w, proj_b, l0_qw, l0_qb, l0_kw, l0_kb, l0_vw,
           l0_vb, l0_sw, l0_sb, l0_bw, l0_ln_g, l0_ln_b, l1_qw, l1_qb, l1_kw,
           l1_kb, l1_vw, l1_vb, l1_sw, l1_sb, l1_bw, l1_ln_g, l1_ln_b,
           cls_w1, cls_b1, cls_w2, cls_b2):
    pad = NP - N
    xi = jnp.pad(x.astype(jnp.int32), (0, pad))
    dep3 = jnp.pad(node_depth.astype(jnp.int32), (0, pad)).reshape(
        NBLK, 1, RB)
    chi3 = jnp.pad(child_index.astype(jnp.int32), (0, pad)).reshape(
        NBLK, 1, RB)
    batch3 = jnp.pad(batch.astype(jnp.int32), (0, pad),
                     constant_values=G).reshape(NBLK, 1, RB)
    srcf = edge_index[0].astype(jnp.int32)
    dstf = edge_index[1].astype(jnp.int32)
    src = srcf.reshape(NW, NREFILL, IDXB, CH)
    dst = dstf.reshape(NW, NREFILL, IDXB, CH)
    srcg = srcf.reshape(NW, AGG_NREFILL, AGG_IDXB, CH)
    dstg = dstf.reshape(NW, AGG_NREFILL, AGG_IDXB, CH)

    nfa = _embed(xi, emb_node)

    h, qp0, kp0, v0, r0 = _tc_proj0(
        nfa, dep3, chi3, emb_depth, emb_child,
        proj_w, proj_b.reshape(1, -1),
        _perm_w(l0_qw), _perm_b(l0_qb), _perm_w(l0_kw), _perm_b(l0_kb),
        l0_vw, l0_vb.reshape(1, -1), l0_sw, l0_sb.reshape(1, -1))

    ex0, den0 = _attn_ex(_pack_words(qp0), _pack_words(kp0), src, dst)
    outp0 = _attn_agg(v0, ex0, den0[0], den0[1], srcg, dstg)

    h0, qp1, kp1, v1, r1 = _tc_mid(
        outp0[0], outp0[1], r0, l0_bw, l0_ln_g.reshape(1, -1),
        l0_ln_b.reshape(1, -1),
        _perm_w(l1_qw), _perm_b(l1_qb), _perm_w(l1_kw), _perm_b(l1_kb),
        l1_vw, l1_vb.reshape(1, -1), l1_sw, l1_sb.reshape(1, -1))

    ex1, den1 = _attn_ex(_pack_words(qp1), _pack_words(kp1), src, dst)
    outp1 = _attn_agg(v1, ex1, den1[0], den1[1], srcg, dstg)

    psum, pcnt = _tc_pool(outp1[0], outp1[1], r1, h0, l1_bw,
                          l1_ln_g.reshape(1, -1), l1_ln_b.reshape(1, -1),
                          batch3)
    return _tc_cls(psum, pcnt, cls_w1, cls_b1.reshape(1, -1),
                   cls_w2, cls_b2.reshape(1, -1))[0]


# speedup vs baseline: 14.7997x; 1.1281x over previous
import functools
import math

import jax
import jax.numpy as jnp
from jax import lax
from jax.experimental import pallas as pl
from jax.experimental.pallas import tpu as pltpu
from jax.experimental.pallas import tpu_sc as plsc

N = 10000
NP = 10240
E = 320000
EMB = 256
DD = 32
CD = 32
HID = 128
H = 8
G = 64
QK = H * HID
QKP = QK // 2

NC = 2
NS = 16
NW = NC * NS
EW = E // NW
CH = 16
NCHUNK = EW // CH
IDXB = 25
NREFILL = NCHUNK // IDXB
AGG_IDXB = 5
AGG_NREFILL = NCHUNK // AGG_IDXB
ROWS_T = NP // NS
NROW_W = NP // NW

LANES = [0, 4, 1, 5, 2, 6, 3, 7]

PERM128 = tuple([32 * (p // 16) + 2 * (p % 16) for p in range(64)]
                + [32 * (p // 16) + 2 * (p % 16) + 1 for p in range(64)])

_SC_MESH = dict(core_axis_name="c", subcore_axis_name="s",
                num_cores=NC, num_subcores=NS)

F32 = jnp.float32
BF16 = jnp.bfloat16


def _sds(shape, dtype=F32):
    return jax.ShapeDtypeStruct(shape, dtype)


@functools.partial(
    pl.kernel,
    out_type=_sds((NP, EMB)),
    mesh=plsc.VectorSubcoreMesh(**_SC_MESH),
    scratch_types=[
        pltpu.VMEM((64,), jnp.int32),
        pltpu.VMEM((64, EMB), F32),
        pltpu.SemaphoreType.DMA,
    ],
)
def _embed(x_h, tn_h, nfa_h, ib, nb, sem):
    cid = lax.axis_index("c")
    sid = lax.axis_index("s")
    wid = sid * NC + cid
    base0 = wid * NROW_W

    def chunk(i, _):
        base = base0 + i * 64
        pltpu.sync_copy(x_h.at[pl.ds(base, 64)], ib)
        pltpu.async_copy(tn_h.at[ib], nb, sem).wait()
        pltpu.sync_copy(nb, nfa_h.at[pl.ds(base, 64)])
        return _

    lax.fori_loop(0, NROW_W // 64, chunk, None)


@functools.partial(
    pl.kernel,
    out_type=(_sds((E, 16)), _sds((NC, NP, HID))),
    mesh=plsc.VectorSubcoreMesh(**_SC_MESH),
    scratch_types=[
        pltpu.VMEM((IDXB, CH), jnp.int32),
        pltpu.VMEM((IDXB, CH), jnp.int32),
        pltpu.VMEM((CH, QKP), F32),
        pltpu.VMEM((CH, QKP), F32),
        pltpu.VMEM((CH, QKP), F32),
        pltpu.VMEM((CH, QKP), F32),
        pltpu.VMEM((CH, 16), F32),
        pltpu.VMEM((CH, HID), F32),
        pltpu.VMEM((16, HID), F32),
        pltpu.VMEM((32,), F32),
        pltpu.VMEM_SHARED((NP, HID), F32),
        pltpu.SemaphoreType.DMA,
        pltpu.SemaphoreType.DMA,
        pltpu.SemaphoreType.DMA,
        pltpu.SemaphoreType.DMA,
    ],
)
def _attn_ex(qp_h, kp_h, srcr_h, dstr_h, ex_h, den_h,
             src_i, dst_i, qba, kba, qbb, kbb, exb, exd, zb, tmp, den_sh,
             sqa, ska, sqb, skb):
    cid = lax.axis_index("c")
    sid = lax.axis_index("s")
    wid = sid * NC + cid
    zv = jnp.zeros((16,), F32)

    def zrow(i, _):
        for t in range(HID // 16):
            zb[i, pl.ds(16 * t, 16)] = zv
        return _

    lax.fori_loop(0, 16, zrow, None)

    def zex(e, _):
        for t in range(HID // 16):
            exd[e, pl.ds(16 * t, 16)] = zv
        return _

    lax.fori_loop(0, CH, zex, None)

    def zcopy(t, _):
        pltpu.sync_copy(zb, den_sh.at[pl.ds(sid * ROWS_T + t * 16, 16)])
        return _

    lax.fori_loop(0, ROWS_T // 16, zcopy, None)
    plsc.subcore_barrier()

    scale = 1.0 / math.sqrt(float(HID))
    ebase = wid * EW
    mask_hi = jnp.int32(-65536)

    def split(v16):
        wi = lax.bitcast_convert_type(v16, jnp.int32)
        ev = lax.bitcast_convert_type(lax.shift_left(wi, 16), F32)
        od = lax.bitcast_convert_type(lax.bitwise_and(wi, mask_hi), F32)
        return ev, od

    def start(j, qb, kb, sq, sk):
        pltpu.async_copy(qp_h.at[dst_i.at[j]], qb, sq)
        pltpu.async_copy(kp_h.at[src_i.at[j]], kb, sk)

    def wait(qb, kb, sq, sk):
        pltpu.make_async_copy(qp_h.at[dst_i.at[0]], qb, sq).wait()
        pltpu.make_async_copy(kp_h.at[src_i.at[0]], kb, sk).wait()

    def compute(rj, j, qb, kb):
        def edge(e, _):
            acc_e = None
            acc_o = None
            for t in range(QKP // 16):
                qe, qo = split(qb[e, pl.ds(16 * t, 16)])
                ke, ko = split(kb[e, pl.ds(16 * t, 16)])
                if acc_e is None:
                    acc_e = qe * ke
                    acc_o = qo * ko
                else:
                    acc_e = acc_e + qe * ke
                    acc_o = acc_o + qo * ko
            folded = []
            for a in (acc_e, acc_o):
                tmp[pl.ds(0, 16)] = a
                tmp[pl.ds(16, 16)] = a
                s1 = a + tmp[pl.ds(8, 16)]
                tmp[pl.ds(0, 16)] = s1
                tmp[pl.ds(16, 16)] = s1
                folded.append(s1 + tmp[pl.ds(4, 16)])
            tmp[pl.ds(0, 16)] = folded[0]
            tmp[pl.ds(4, 16)] = folded[1]
            ev = jnp.exp(tmp[pl.ds(0, 16)] * scale)
            exb[e, :] = ev
            exd[e, pl.ds(0, 16)] = ev
            return _

        lax.fori_loop(0, CH, edge, None)
        pltpu.sync_copy(exb,
                        ex_h.at[pl.ds(ebase + (rj * IDXB + j) * CH, CH)])
        pltpu.sync_copy(exd, den_sh.at[dst_i.at[j]], add=True)

    def refill(rj, _):
        pltpu.sync_copy(srcr_h.at[wid, rj], src_i)
        pltpu.sync_copy(dstr_h.at[wid, rj], dst_i)
        start(0, qba, kba, sqa, ska)

        def pair(jj, _):
            j0 = 2 * jj
            start(j0 + 1, qbb, kbb, sqb, skb)
            wait(qba, kba, sqa, ska)
            compute(rj, j0, qba, kba)
            start(j0 + 2, qba, kba, sqa, ska)
            wait(qbb, kbb, sqb, skb)
            compute(rj, j0 + 1, qbb, kbb)
            return _

        lax.fori_loop(0, (IDXB - 1) // 2, pair, None)
        wait(qba, kba, sqa, ska)
        compute(rj, IDXB - 1, qba, kba)
        return _

    lax.fori_loop(0, NREFILL, refill, None)
    plsc.subcore_barrier()

    def dump(t, _):
        off = sid * ROWS_T + t * 64
        pltpu.sync_copy(den_sh.at[pl.ds(off, 64)],
                        den_h.at[cid, pl.ds(off, 64)])
        return _

    lax.fori_loop(0, ROWS_T // 64, dump, None)


@functools.partial(
    pl.kernel,
    out_type=_sds((NC, NP, HID)),
    mesh=plsc.VectorSubcoreMesh(**_SC_MESH),
    scratch_types=[
        pltpu.VMEM((AGG_IDXB, CH), jnp.int32),
        pltpu.VMEM((AGG_IDXB, CH), jnp.int32),
        pltpu.VMEM((CH, QKP), F32),
        pltpu.VMEM((CH, QKP), F32),
        pltpu.VMEM((CH, 16), F32),
        pltpu.VMEM((CH, 16), F32),
        pltpu.VMEM((CH, HID), F32),
        pltpu.VMEM((CH, HID), F32),
        pltpu.VMEM((CH, HID), F32),
        pltpu.VMEM((8, HID), F32),
        pltpu.VMEM_SHARED((NP, HID), F32),
        pltpu.SemaphoreType.DMA,
        pltpu.SemaphoreType.DMA,
        pltpu.SemaphoreType.DMA,
        pltpu.SemaphoreType.DMA,
        pltpu.SemaphoreType.DMA,
        pltpu.SemaphoreType.DMA,
    ],
)
def _attn_agg(v_h, ex_h, den_h, srcr_h, dstr_h, outp_h,
              src_i, dst_i, vba, vbb, exba, exbb, dba, dbb, mb, zb, out_sh,
              sva, svb, sda, sdb, sea, seb):
    cid = lax.axis_index("c")
    sid = lax.axis_index("s")
    wid = sid * NC + cid
    zv = jnp.zeros((16,), F32)
    mask_hi = jnp.int32(-65536)

    def split(v16):
        wi = lax.bitcast_convert_type(v16, jnp.int32)
        ev = lax.bitcast_convert_type(lax.shift_left(wi, 16), F32)
        od = lax.bitcast_convert_type(lax.bitwise_and(wi, mask_hi), F32)
        return ev, od

    def zrow(i, _):
        for t in range(HID // 16):
            zb[i, pl.ds(16 * t, 16)] = zv
        return _

    lax.fori_loop(0, 8, zrow, None)

    def zcopy(t, _):
        pltpu.sync_copy(zb, out_sh.at[pl.ds(sid * ROWS_T + t * 8, 8)])
        return _

    lax.fori_loop(0, ROWS_T // 8, zcopy, None)
    plsc.subcore_barrier()

    ebase = wid * EW

    def start(rj, j, vb, db, eb, sv, sd, se):
        pltpu.async_copy(v_h.at[src_i.at[j]], vb, sv)
        pltpu.async_copy(den_h.at[dst_i.at[j]], db, sd)
        pltpu.async_copy(
            ex_h.at[pl.ds(ebase + (rj * AGG_IDXB + j) * CH, CH)], eb, se)

    def wait(vb, db, eb, sv, sd, se):
        pltpu.make_async_copy(v_h.at[src_i.at[0]], vb, sv).wait()
        pltpu.make_async_copy(den_h.at[dst_i.at[0]], db, sd).wait()
        pltpu.make_async_copy(ex_h.at[pl.ds(0, CH)], eb, se).wait()

    def compute(j, vb, db, eb):
        def edge(e, _):
            denv = db[e, pl.ds(0, 16)]
            alpha = eb[e, :] / jnp.maximum(denv, 1e-16)
            msgs_e = None
            msgs_o = None
            for hh in range(H):
                ah = alpha[LANES[hh]]
                hb = hh * (HID // 2)
                if msgs_e is None:
                    pairs = [split(vb[e, pl.ds(hb + 16 * u, 16)])
                             for u in range(HID // 32)]
                    msgs_e = [ah * pe for pe, _po in pairs]
                    msgs_o = [ah * po for _pe, po in pairs]
                else:
                    for u in range(HID // 32):
                        ve, vo = split(vb[e, pl.ds(hb + 16 * u, 16)])
                        msgs_e[u] = msgs_e[u] + ah * ve
                        msgs_o[u] = msgs_o[u] + ah * vo
            for u in range(HID // 32):
                mb[e, pl.ds(16 * u, 16)] = msgs_e[u]
                mb[e, pl.ds(64 + 16 * u, 16)] = msgs_o[u]
            return _

        lax.fori_loop(0, CH, edge, None)
        pltpu.sync_copy(mb, out_sh.at[dst_i.at[j]], add=True)

    def refill(rj, _):
        pltpu.sync_copy(srcr_h.at[wid, rj], src_i)
        pltpu.sync_copy(dstr_h.at[wid, rj], dst_i)
        start(rj, 0, vba, dba, exba, sva, sda, sea)

        def pair(jj, _):
            j0 = 2 * jj
            start(rj, j0 + 1, vbb, dbb, exbb, svb, sdb, seb)
            wait(vba, dba, exba, sva, sda, sea)
            compute(j0, vba, dba, exba)
            start(rj, j0 + 2, vba, dba, exba, sva, sda, sea)
            wait(vbb, dbb, exbb, svb, sdb, seb)
            compute(j0 + 1, vbb, dbb, exbb)
            return _

        lax.fori_loop(0, (AGG_IDXB - 1) // 2, pair, None)
        wait(vba, dba, exba, sva, sda, sea)
        compute(AGG_IDXB - 1, vba, dba, exba)
        return _

    lax.fori_loop(0, AGG_NREFILL, refill, None)
    plsc.subcore_barrier()

    def dump(t, _):
        off = sid * ROWS_T + t * 64
        pltpu.sync_copy(out_sh.at[pl.ds(off, 64)],
                        outp_h.at[cid, pl.ds(off, 64)])
        return _

    lax.fori_loop(0, ROWS_T // 64, dump, None)


RB = 512
NBLK = NP // RB

DDEP = 51
DCHI = 21


def _onehot_lookup(ids3, table, n_ids):
    ids = ids3[0]
    oh = (lax.broadcasted_iota(jnp.int32, (n_ids, 1), 0) == ids).astype(F32)
    return lax.dot_general(oh, table, (((0,), (0,)), ((), ())),
                           preferred_element_type=F32)


def _tc_proj0_body(nfa, dep3, chi3, td, tc, pw, pb, qw, qb, kw, kb, vw, vb,
                   sw, sb, h_o, qp_o, kp_o, v_o, r_o):
    pwv = pw[...]
    nfb = _onehot_lookup(dep3, td[...], DDEP)
    nfc = _onehot_lookup(chi3, tc[...], DCHI)
    h = (jnp.dot(nfa[...], pwv[0:EMB], preferred_element_type=F32)
         + jnp.dot(nfb, pwv[EMB:EMB + DD], preferred_element_type=F32)
         + jnp.dot(nfc, pwv[EMB + DD:], preferred_element_type=F32)
         + pb[...])
    h = jnp.maximum(h, 0.0)
    h_o[...] = h
    qp_o[...] = (jnp.dot(h, qw[...], preferred_element_type=F32)
                 + qb[...]).astype(BF16)
    kp_o[...] = (jnp.dot(h, kw[...], preferred_element_type=F32)
                 + kb[...]).astype(BF16)
    v_o[...] = (jnp.dot(h, vw[...], preferred_element_type=F32)
                + vb[...]).astype(BF16)
    r_o[...] = jnp.dot(h, sw[...], preferred_element_type=F32) + sb[...]


def _pack_words(a):
    return lax.bitcast_convert_type(a.reshape(NP, QKP, 2), F32)


def _full(shape):
    nd = len(shape)
    return pl.BlockSpec(shape, lambda i: (0,) * nd)


def _rows(cols):
    return pl.BlockSpec((RB, cols), lambda i: (i, 0))


def _tc_proj0(nfa, dep3, chi3, td, tc, pw, pb, qw, qb, kw, kb, vw, vb,
              sw, sb):
    return pl.pallas_call(
        _tc_proj0_body,
        grid=(NBLK,),
        in_specs=[_rows(EMB),
                  pl.BlockSpec((1, 1, RB), lambda i: (i, 0, 0)),
                  pl.BlockSpec((1, 1, RB), lambda i: (i, 0, 0)),
                  _full((DDEP, DD)), _full((DCHI, CD)),
                  _full((EMB + DD + CD, EMB)), _full((1, EMB)),
                  _full((EMB, QK)), _full((1, QK)),
                  _full((EMB, QK)), _full((1, QK)),
                  _full((EMB, QK)), _full((1, QK)),
                  _full((EMB, HID)), _full((1, HID))],
        out_specs=[_rows(EMB), _rows(QK), _rows(QK), _rows(QK),
                   _rows(HID)],
        out_shape=[_sds((NP, EMB)), _sds((NP, QK), BF16),
                   _sds((NP, QK), BF16), _sds((NP, QK), BF16),
                   _sds((NP, HID))],
    )(nfa, dep3, chi3, td, tc, pw, pb, qw, qb, kw, kb, vw, vb, sw, sb)


def _gate(o, rr, bwv):
    z = (jnp.dot(o, bwv[0:HID], preferred_element_type=F32)
         + jnp.dot(rr, bwv[HID:2 * HID], preferred_element_type=F32)
         + jnp.dot(o - rr, bwv[2 * HID:], preferred_element_type=F32))
    g = jax.nn.sigmoid(z)
    return g * rr + (1.0 - g) * o


def _ln_relu(hx, lng, lnb):
    mu = jnp.mean(hx, axis=1, keepdims=True)
    var = jnp.mean((hx - mu) * (hx - mu), axis=1, keepdims=True)
    y = (hx - mu) / jnp.sqrt(var + 1e-5) * lng + lnb
    return jnp.maximum(y, 0.0)


def _tc_mid_body(p0, p1, r, bw, lng, lnb, qw, qb, kw, kb, vw, vb, sw, sb,
                 h0_o, qp_o, kp_o, v_o, r_o):
    o = (p0[...] + p1[...]) * (1.0 / H)
    hx = _gate(o, r[...], bw[...])
    y = _ln_relu(hx, lng[...], lnb[...])
    h0_o[...] = y
    qp_o[...] = (jnp.dot(y, qw[...], preferred_element_type=F32)
                 + qb[...]).astype(BF16)
    kp_o[...] = (jnp.dot(y, kw[...], preferred_element_type=F32)
                 + kb[...]).astype(BF16)
    v_o[...] = (jnp.dot(y, vw[...], preferred_element_type=F32)
                + vb[...]).astype(BF16)
    r_o[...] = jnp.dot(y, sw[...], preferred_element_type=F32) + sb[...]


def _tc_mid(p0, p1, r, bw, lng, lnb, qw, qb, kw, kb, vw, vb, sw, sb):
    return pl.pallas_call(
        _tc_mid_body,
        grid=(NBLK,),
        in_specs=[_rows(HID), _rows(HID), _rows(HID),
                  _full((3 * HID, 1)), _full((1, HID)), _full((1, HID)),
                  _full((HID, QK)), _full((1, QK)),
                  _full((HID, QK)), _full((1, QK)),
                  _full((HID, QK)), _full((1, QK)),
                  _full((HID, HID)), _full((1, HID))],
        out_specs=[_rows(HID), _rows(QK), _rows(QK), _rows(QK),
                   _rows(HID)],
        out_shape=[_sds((NP, HID)), _sds((NP, QK), BF16),
                   _sds((NP, QK), BF16), _sds((NP, QK), BF16),
                   _sds((NP, HID))],
    )(p0, p1, r, bw, lng, lnb, qw, qb, kw, kb, vw, vb, sw, sb)


def _tc_pool_body(p0, p1, r, h0, bw, lng, lnb, batch3, psum_o, pcnt_o):
    i = pl.program_id(0)
    o = (p0[...] + p1[...]) * (1.0 / H)
    hx = _gate(o, r[...], bw[...]) + h0[...]
    y = _ln_relu(hx, lng[...], lnb[...])
    b = batch3[0]
    seg = lax.broadcasted_iota(jnp.int32, (G, 1), 0)
    oh = (b == seg).astype(F32)

    @pl.when(i == 0)
    def _():
        psum_o[...] = jnp.zeros_like(psum_o)
        pcnt_o[...] = jnp.zeros_like(pcnt_o)

    psum_o[...] += jnp.dot(oh, y, preferred_element_type=F32)
    pcnt_o[...] += jnp.broadcast_to(jnp.sum(oh, axis=1, keepdims=True),
                                    (G, HID))


def _tc_pool(p0, p1, r, h0, bw, lng, lnb, batch3):
    return pl.pallas_call(
        _tc_pool_body,
        grid=(NBLK,),
        in_specs=[_rows(HID), _rows(HID), _rows(HID), _rows(HID),
                  _full((3 * HID, 1)), _full((1, HID)), _full((1, HID)),
                  pl.BlockSpec((1, 1, RB), lambda i: (i, 0, 0))],
        out_specs=[_full((G, HID)), _full((G, HID))],
        out_shape=[_sds((G, HID)), _sds((G, HID))],
    )(p0, p1, r, h0, bw, lng, lnb, batch3)


def _tc_cls_body(psum, pcnt, w1, b1, w2, b2, out_o):
    pooled = psum[...] / jnp.maximum(pcnt[...], 1.0)
    hc = jnp.maximum(jnp.dot(pooled, w1[...], preferred_element_type=F32)
                     + b1[...], 0.0)
    out_o[...] = jnp.dot(hc, w2[...], preferred_element_type=F32) + b2[...]


def _tc_cls(psum, pcnt, w1, b1, w2, b2):
    return pl.pallas_call(
        _tc_cls_body,
        grid=(1,),
        in_specs=[_full((G, HID)), _full((G, HID)),
                  _full((HID, HID // 2)), _full((1, HID // 2)),
                  _full((HID // 2, 1)), _full((1, 1))],
        out_specs=[_full((G, 1))],
        out_shape=[_sds((G, 1))],
    )(psum, pcnt, w1, b1, w2, b2)


def _tc_densum_body(d0, d1, o_o):
    o_o[...] = d0[...] + d1[...]


def _tc_densum(d0, d1):
    return pl.pallas_call(
        _tc_densum_body,
        grid=(NBLK,),
        in_specs=[_rows(HID), _rows(HID)],
        out_specs=[_rows(HID)],
        out_shape=[_sds((NP, HID))],
    )(d0, d1)[0]


def _perm_w(w):
    return w.reshape(-1, H, HID).transpose(0, 2, 1).reshape(-1, H * HID)


def _perm_b(b):
    return b.reshape(H, HID).T.reshape(1, -1)


def kernel(x, edge_index, batch, node_depth, child_index, emb_node, emb_depth,
           emb_child, proj_w, proj_b, l0_qw, l0_qb, l0_kw, l0_kb, l0_vw,
           l0_vb, l0_sw, l0_sb, l0_bw, l0_ln_g, l0_ln_b, l1_qw, l1_qb, l1_kw,
           l1_kb, l1_vw, l1_vb, l1_sw, l1_sb, l1_bw, l1_ln_g, l1_ln_b,
           cls_w1, cls_b1, cls_w2, cls_b2):
    pad = NP - N
    xi = jnp.pad(x.astype(jnp.int32), (0, pad))
    dep3 = jnp.pad(node_depth.astype(jnp.int32), (0, pad)).reshape(
        NBLK, 1, RB)
    chi3 = jnp.pad(child_index.astype(jnp.int32), (0, pad)).reshape(
        NBLK, 1, RB)
    batch3 = jnp.pad(batch.astype(jnp.int32), (0, pad),
                     constant_values=G).reshape(NBLK, 1, RB)
    srcf = edge_index[0].astype(jnp.int32)
    dstf = edge_index[1].astype(jnp.int32)
    src = srcf.reshape(NW, NREFILL, IDXB, CH)
    dst = dstf.reshape(NW, NREFILL, IDXB, CH)
    srcg = srcf.reshape(NW, AGG_NREFILL, AGG_IDXB, CH)
    dstg = dstf.reshape(NW, AGG_NREFILL, AGG_IDXB, CH)

    nfa = _embed(xi, emb_node)

    P = jnp.array(PERM128, dtype=jnp.int32)

    def permc(w):
        return w[:, P]

    def permr(w):
        return w[P]

    def perm_bw(bw):
        return jnp.concatenate([bw[0:HID][P], bw[HID:2 * HID][P],
                                bw[2 * HID:][P]], axis=0)

    h, qp0, kp0, v0, r0 = _tc_proj0(
        nfa, dep3, chi3, emb_depth, emb_child,
        proj_w, proj_b.reshape(1, -1),
        _perm_w(l0_qw), _perm_b(l0_qb), _perm_w(l0_kw), _perm_b(l0_kb),
        l0_vw, l0_vb.reshape(1, -1), permc(l0_sw),
        l0_sb[P].reshape(1, -1))

    ex0, den0 = _attn_ex(_pack_words(qp0), _pack_words(kp0), src, dst)
    outp0 = _attn_agg(_pack_words(v0), ex0,
                      _tc_densum(den0[0], den0[1]), srcg, dstg)

    h0, qp1, kp1, v1, r1 = _tc_mid(
        outp0[0], outp0[1], r0, perm_bw(l0_bw),
        l0_ln_g[P].reshape(1, -1), l0_ln_b[P].reshape(1, -1),
        _perm_w(permr(l1_qw)), _perm_b(l1_qb),
        _perm_w(permr(l1_kw)), _perm_b(l1_kb),
        permr(l1_vw), l1_vb.reshape(1, -1), permc(permr(l1_sw)),
        l1_sb[P].reshape(1, -1))

    ex1, den1 = _attn_ex(_pack_words(qp1), _pack_words(kp1), src, dst)
    outp1 = _attn_agg(_pack_words(v1), ex1,
                      _tc_densum(den1[0], den1[1]), srcg, dstg)

    psum, pcnt = _tc_pool(outp1[0], outp1[1], r1, h0, perm_bw(l1_bw),
                          l1_ln_g[P].reshape(1, -1),
                          l1_ln_b[P].reshape(1, -1), batch3)
    return _tc_cls(psum, pcnt, permr(cls_w1), cls_b1.reshape(1, -1),
                   cls_w2, cls_b2.reshape(1, -1))[0]


# speedup vs baseline: 21.5055x; 1.4531x over previous
import functools
import math

import jax
import jax.numpy as jnp
from jax import lax
from jax.experimental import pallas as pl
from jax.experimental.pallas import tpu as pltpu
from jax.experimental.pallas import tpu_sc as plsc

N = 10000
NP = 10240
E = 320000
EMB = 256
DD = 32
CD = 32
HID = 128
H = 8
G = 64
QK = H * HID
QKP = QK // 2

NC = 2
NS = 16
NW = NC * NS
EW = E // NW
CH = 16
NCHUNK = EW // CH
IDXB = 25
NREFILL = NCHUNK // IDXB
AGG_IDXB = 25
AGG_NREFILL = NCHUNK // AGG_IDXB
ROWS_T = NP // NS
NROW_W = NP // NW

LANES = [0, 4, 1, 5, 2, 6, 3, 7]

PERM128 = tuple([32 * (p // 16) + 2 * (p % 16) for p in range(64)]
                + [32 * (p // 16) + 2 * (p % 16) + 1 for p in range(64)])

_SC_MESH = dict(core_axis_name="c", subcore_axis_name="s",
                num_cores=NC, num_subcores=NS)

F32 = jnp.float32
BF16 = jnp.bfloat16


def _sds(shape, dtype=F32):
    return jax.ShapeDtypeStruct(shape, dtype)


@functools.partial(
    pl.kernel,
    out_type=_sds((NP, EMB)),
    mesh=plsc.VectorSubcoreMesh(**_SC_MESH),
    scratch_types=[
        pltpu.VMEM((64,), jnp.int32),
        pltpu.VMEM((64, EMB), F32),
        pltpu.SemaphoreType.DMA,
    ],
)
def _embed(x_h, tn_h, nfa_h, ib, nb, sem):
    cid = lax.axis_index("c")
    sid = lax.axis_index("s")
    wid = sid * NC + cid
    base0 = wid * NROW_W

    def chunk(i, _):
        base = base0 + i * 64
        pltpu.sync_copy(x_h.at[pl.ds(base, 64)], ib)
        pltpu.async_copy(tn_h.at[ib], nb, sem).wait()
        pltpu.sync_copy(nb, nfa_h.at[pl.ds(base, 64)])
        return _

    lax.fori_loop(0, NROW_W // 64, chunk, None)


@functools.partial(
    pl.kernel,
    out_type=(_sds((E, 16)), _sds((NC, NP, HID))),
    mesh=plsc.VectorSubcoreMesh(**_SC_MESH),
    scratch_types=[
        pltpu.VMEM((IDXB, CH), jnp.int32),
        pltpu.VMEM((IDXB, CH), jnp.int32),
        pltpu.VMEM((CH, QKP), F32),
        pltpu.VMEM((CH, QKP), F32),
        pltpu.VMEM((CH, QKP), F32),
        pltpu.VMEM((CH, QKP), F32),
        pltpu.VMEM((CH, 16), F32),
        pltpu.VMEM((CH, HID), F32),
        pltpu.VMEM((16, HID), F32),
        pltpu.VMEM((32,), F32),
        pltpu.VMEM_SHARED((NP, HID), F32),
        pltpu.SemaphoreType.DMA,
        pltpu.SemaphoreType.DMA,
        pltpu.SemaphoreType.DMA,
        pltpu.SemaphoreType.DMA,
    ],
)
def _attn_ex(qp_h, kp_h, srcr_h, dstr_h, ex_h, den_h,
             src_i, dst_i, qba, kba, qbb, kbb, exb, exd, zb, tmp, den_sh,
             sqa, ska, sqb, skb):
    cid = lax.axis_index("c")
    sid = lax.axis_index("s")
    wid = sid * NC + cid
    zv = jnp.zeros((16,), F32)

    def zrow(i, _):
        for t in range(HID // 16):
            zb[i, pl.ds(16 * t, 16)] = zv
        return _

    lax.fori_loop(0, 16, zrow, None)

    def zex(e, _):
        for t in range(HID // 16):
            exd[e, pl.ds(16 * t, 16)] = zv
        return _

    lax.fori_loop(0, CH, zex, None)

    def zcopy(t, _):
        pltpu.sync_copy(zb, den_sh.at[pl.ds(sid * ROWS_T + t * 16, 16)])
        return _

    lax.fori_loop(0, ROWS_T // 16, zcopy, None)
    plsc.subcore_barrier()

    scale = 1.0 / math.sqrt(float(HID))
    ebase = wid * EW
    mask_hi = jnp.int32(-65536)

    def split(v16):
        wi = lax.bitcast_convert_type(v16, jnp.int32)
        ev = lax.bitcast_convert_type(lax.shift_left(wi, 16), F32)
        od = lax.bitcast_convert_type(lax.bitwise_and(wi, mask_hi), F32)
        return ev, od

    def start(j, qb, kb, sq, sk):
        pltpu.async_copy(qp_h.at[dst_i.at[j]], qb, sq)
        pltpu.async_copy(kp_h.at[src_i.at[j]], kb, sk)

    def wait(qb, kb, sq, sk):
        pltpu.make_async_copy(qp_h.at[dst_i.at[0]], qb, sq).wait()
        pltpu.make_async_copy(kp_h.at[src_i.at[0]], kb, sk).wait()

    def compute(rj, j, qb, kb):
        def edge(e, _):
            acc_e = None
            acc_o = None
            for t in range(QKP // 16):
                qe, qo = split(qb[e, pl.ds(16 * t, 16)])
                ke, ko = split(kb[e, pl.ds(16 * t, 16)])
                if acc_e is None:
                    acc_e = qe * ke
                    acc_o = qo * ko
                else:
                    acc_e = acc_e + qe * ke
                    acc_o = acc_o + qo * ko
            folded = []
            for a in (acc_e, acc_o):
                tmp[pl.ds(0, 16)] = a
                tmp[pl.ds(16, 16)] = a
                s1 = a + tmp[pl.ds(8, 16)]
                tmp[pl.ds(0, 16)] = s1
                tmp[pl.ds(16, 16)] = s1
                folded.append(s1 + tmp[pl.ds(4, 16)])
            tmp[pl.ds(0, 16)] = folded[0]
            tmp[pl.ds(4, 16)] = folded[1]
            ev = jnp.exp(tmp[pl.ds(0, 16)] * scale)
            exb[e, :] = ev
            exd[e, pl.ds(0, 16)] = ev
            return _

        lax.fori_loop(0, CH, edge, None)
        pltpu.sync_copy(exb,
                        ex_h.at[pl.ds(ebase + (rj * IDXB + j) * CH, CH)])
        pltpu.sync_copy(exd, den_sh.at[dst_i.at[j]], add=True)

    def refill(rj, _):
        pltpu.sync_copy(srcr_h.at[wid, rj], src_i)
        pltpu.sync_copy(dstr_h.at[wid, rj], dst_i)
        start(0, qba, kba, sqa, ska)

        def pair(jj, _):
            j0 = 2 * jj
            start(j0 + 1, qbb, kbb, sqb, skb)
            wait(qba, kba, sqa, ska)
            compute(rj, j0, qba, kba)
            start(j0 + 2, qba, kba, sqa, ska)
            wait(qbb, kbb, sqb, skb)
            compute(rj, j0 + 1, qbb, kbb)
            return _

        lax.fori_loop(0, (IDXB - 1) // 2, pair, None)
        wait(qba, kba, sqa, ska)
        compute(rj, IDXB - 1, qba, kba)
        return _

    lax.fori_loop(0, NREFILL, refill, None)
    plsc.subcore_barrier()

    def dump(t, _):
        off = sid * ROWS_T + t * 64
        pltpu.sync_copy(den_sh.at[pl.ds(off, 64)],
                        den_h.at[cid, pl.ds(off, 64)])
        return _

    lax.fori_loop(0, ROWS_T // 64, dump, None)


@functools.partial(
    pl.kernel,
    out_type=_sds((NC, NP, HID)),
    mesh=plsc.VectorSubcoreMesh(**_SC_MESH),
    scratch_types=[
        pltpu.VMEM((AGG_IDXB, CH), jnp.int32),
        pltpu.VMEM((AGG_IDXB, CH), jnp.int32),
        pltpu.VMEM((CH, QKP), F32),
        pltpu.VMEM((CH, QKP), F32),
        pltpu.VMEM((CH, 16), F32),
        pltpu.VMEM((CH, 16), F32),
        pltpu.VMEM((CH, HID), F32),
        pltpu.VMEM((CH, HID), F32),
        pltpu.VMEM((CH, HID), F32),
        pltpu.VMEM((8, HID), F32),
        pltpu.VMEM_SHARED((NP, HID), F32),
        pltpu.SemaphoreType.DMA,
        pltpu.SemaphoreType.DMA,
        pltpu.SemaphoreType.DMA,
        pltpu.SemaphoreType.DMA,
        pltpu.SemaphoreType.DMA,
        pltpu.SemaphoreType.DMA,
    ],
)
def _attn_agg(v_h, ex_h, den_h, srcr_h, dstr_h, outp_h,
              src_i, dst_i, vba, vbb, exba, exbb, dba, dbb, mb, zb, out_sh,
              sva, svb, sda, sdb, sea, seb):
    cid = lax.axis_index("c")
    sid = lax.axis_index("s")
    wid = sid * NC + cid
    zv = jnp.zeros((16,), F32)
    mask_hi = jnp.int32(-65536)

    def split(v16):
        wi = lax.bitcast_convert_type(v16, jnp.int32)
        ev = lax.bitcast_convert_type(lax.shift_left(wi, 16), F32)
        od = lax.bitcast_convert_type(lax.bitwise_and(wi, mask_hi), F32)
        return ev, od

    def zrow(i, _):
        for t in range(HID // 16):
            zb[i, pl.ds(16 * t, 16)] = zv
        return _

    lax.fori_loop(0, 8, zrow, None)

    def zcopy(t, _):
        pltpu.sync_copy(zb, out_sh.at[pl.ds(sid * ROWS_T + t * 8, 8)])
        return _

    lax.fori_loop(0, ROWS_T // 8, zcopy, None)
    plsc.subcore_barrier()

    ebase = wid * EW

    def start(rj, j, vb, db, eb, sv, sd, se):
        pltpu.async_copy(v_h.at[src_i.at[j]], vb, sv)
        pltpu.async_copy(den_h.at[dst_i.at[j]], db, sd)
        pltpu.async_copy(
            ex_h.at[pl.ds(ebase + (rj * AGG_IDXB + j) * CH, CH)], eb, se)

    def wait(vb, db, eb, sv, sd, se):
        pltpu.make_async_copy(v_h.at[src_i.at[0]], vb, sv).wait()
        pltpu.make_async_copy(den_h.at[dst_i.at[0]], db, sd).wait()
        pltpu.make_async_copy(ex_h.at[pl.ds(0, CH)], eb, se).wait()

    def compute(j, vb, db, eb):
        def edge(e, _):
            denv = db[e, pl.ds(0, 16)]
            alpha = eb[e, :] / jnp.maximum(denv, 1e-16)
            msgs_e = None
            msgs_o = None
            for hh in range(H):
                ah = alpha[LANES[hh]]
                hb = hh * (HID // 2)
                if msgs_e is None:
                    pairs = [split(vb[e, pl.ds(hb + 16 * u, 16)])
                             for u in range(HID // 32)]
                    msgs_e = [ah * pe for pe, _po in pairs]
                    msgs_o = [ah * po for _pe, po in pairs]
                else:
                    for u in range(HID // 32):
                        ve, vo = split(vb[e, pl.ds(hb + 16 * u, 16)])
                        msgs_e[u] = msgs_e[u] + ah * ve
                        msgs_o[u] = msgs_o[u] + ah * vo
            for u in range(HID // 32):
                mb[e, pl.ds(16 * u, 16)] = msgs_e[u]
                mb[e, pl.ds(64 + 16 * u, 16)] = msgs_o[u]
            return _

        lax.fori_loop(0, CH, edge, None)
        pltpu.sync_copy(mb, out_sh.at[dst_i.at[j]], add=True)

    def refill(rj, _):
        pltpu.sync_copy(srcr_h.at[wid, rj], src_i)
        pltpu.sync_copy(dstr_h.at[wid, rj], dst_i)
        start(rj, 0, vba, dba, exba, sva, sda, sea)

        def pair(jj, _):
            j0 = 2 * jj
            start(rj, j0 + 1, vbb, dbb, exbb, svb, sdb, seb)
            wait(vba, dba, exba, sva, sda, sea)
            compute(j0, vba, dba, exba)
            start(rj, j0 + 2, vba, dba, exba, sva, sda, sea)
            wait(vbb, dbb, exbb, svb, sdb, seb)
            compute(j0 + 1, vbb, dbb, exbb)
            return _

        lax.fori_loop(0, (AGG_IDXB - 1) // 2, pair, None)
        wait(vba, dba, exba, sva, sda, sea)
        compute(AGG_IDXB - 1, vba, dba, exba)
        return _

    lax.fori_loop(0, AGG_NREFILL, refill, None)
    plsc.subcore_barrier()

    def dump(t, _):
        off = sid * ROWS_T + t * 64
        pltpu.sync_copy(out_sh.at[pl.ds(off, 64)],
                        outp_h.at[cid, pl.ds(off, 64)])
        return _

    lax.fori_loop(0, ROWS_T // 64, dump, None)


RB = 512
NBLK = NP // RB

DDEP = 51
DCHI = 21


def _pack2(a, b):
    ai = lax.bitcast_convert_type(a, jnp.int32)
    bi = lax.bitcast_convert_type(b, jnp.int32)
    ar = ai + 32767 + lax.bitwise_and(lax.shift_right_logical(ai, 16), 1)
    br = bi + 32767 + lax.bitwise_and(lax.shift_right_logical(bi, 16), 1)
    lo = lax.shift_right_logical(ar, 16)
    hi = lax.bitwise_and(br, jnp.int32(-65536))
    return lax.bitcast_convert_type(lax.bitwise_or(lo, hi), F32)


def _onehot_lookup(ids3, table, n_ids):
    ids = ids3[0]
    oh = (lax.broadcasted_iota(jnp.int32, (n_ids, 1), 0) == ids).astype(F32)
    return lax.dot_general(oh, table, (((0,), (0,)), ((), ())),
                           preferred_element_type=F32)


def _tc_proj0_body(nfa, dep3, chi3, td, tc, pw, pb, qw, qb, kw, kb, vw, vb,
                   sw, sb, h_o, qp_o, kp_o, v_o, r_o):
    pwv = pw[...]
    nfb = _onehot_lookup(dep3, td[...], DDEP)
    nfc = _onehot_lookup(chi3, tc[...], DCHI)
    h = (jnp.dot(nfa[...], pwv[0:EMB], preferred_element_type=F32)
         + jnp.dot(nfb, pwv[EMB:EMB + DD], preferred_element_type=F32)
         + jnp.dot(nfc, pwv[EMB + DD:], preferred_element_type=F32)
         + pb[...])
    h = jnp.maximum(h, 0.0)
    h_o[...] = h
    qc = jnp.dot(h, qw[...], preferred_element_type=F32) + qb[...]
    qp_o[...] = _pack2(qc[:, 0:QKP], qc[:, QKP:])
    kc = jnp.dot(h, kw[...], preferred_element_type=F32) + kb[...]
    kp_o[...] = _pack2(kc[:, 0:QKP], kc[:, QKP:])
    vc = jnp.dot(h, vw[...], preferred_element_type=F32) + vb[...]
    v_o[...] = _pack2(vc[:, 0:QKP], vc[:, QKP:])
    r_o[...] = jnp.dot(h, sw[...], preferred_element_type=F32) + sb[...]


def _full(shape):
    nd = len(shape)
    return pl.BlockSpec(shape, lambda i: (0,) * nd)


def _rows(cols):
    return pl.BlockSpec((RB, cols), lambda i: (i, 0))


def _tc_proj0(nfa, dep3, chi3, td, tc, pw, pb, qw, qb, kw, kb, vw, vb,
              sw, sb):
    return pl.pallas_call(
        _tc_proj0_body,
        grid=(NBLK,),
        in_specs=[_rows(EMB),
                  pl.BlockSpec((1, 1, RB), lambda i: (i, 0, 0)),
                  pl.BlockSpec((1, 1, RB), lambda i: (i, 0, 0)),
                  _full((DDEP, DD)), _full((DCHI, CD)),
                  _full((EMB + DD + CD, EMB)), _full((1, EMB)),
                  _full((EMB, QK)), _full((1, QK)),
                  _full((EMB, QK)), _full((1, QK)),
                  _full((EMB, QK)), _full((1, QK)),
                  _full((EMB, HID)), _full((1, HID))],
        out_specs=[_rows(EMB), _rows(QKP), _rows(QKP), _rows(QKP),
                   _rows(HID)],
        out_shape=[_sds((NP, EMB)), _sds((NP, QKP)), _sds((NP, QKP)),
                   _sds((NP, QKP)), _sds((NP, HID))],
    )(nfa, dep3, chi3, td, tc, pw, pb, qw, qb, kw, kb, vw, vb, sw, sb)


def _gate(o, rr, bwv):
    z = (jnp.dot(o, bwv[0:HID], preferred_element_type=F32)
         + jnp.dot(rr, bwv[HID:2 * HID], preferred_element_type=F32)
         + jnp.dot(o - rr, bwv[2 * HID:], preferred_element_type=F32))
    g = jax.nn.sigmoid(z)
    return g * rr + (1.0 - g) * o


def _ln_relu(hx, lng, lnb):
    mu = jnp.mean(hx, axis=1, keepdims=True)
    var = jnp.mean((hx - mu) * (hx - mu), axis=1, keepdims=True)
    y = (hx - mu) / jnp.sqrt(var + 1e-5) * lng + lnb
    return jnp.maximum(y, 0.0)


def _tc_mid_body(p0, p1, r, bw, lng, lnb, qw, qb, kw, kb, vw, vb, sw, sb,
                 h0_o, qp_o, kp_o, v_o, r_o):
    o = (p0[...] + p1[...]) * (1.0 / H)
    hx = _gate(o, r[...], bw[...])
    y = _ln_relu(hx, lng[...], lnb[...])
    h0_o[...] = y
    qc = jnp.dot(y, qw[...], preferred_element_type=F32) + qb[...]
    qp_o[...] = _pack2(qc[:, 0:QKP], qc[:, QKP:])
    kc = jnp.dot(y, kw[...], preferred_element_type=F32) + kb[...]
    kp_o[...] = _pack2(kc[:, 0:QKP], kc[:, QKP:])
    vc = jnp.dot(y, vw[...], preferred_element_type=F32) + vb[...]
    v_o[...] = _pack2(vc[:, 0:QKP], vc[:, QKP:])
    r_o[...] = jnp.dot(y, sw[...], preferred_element_type=F32) + sb[...]


def _tc_mid(p0, p1, r, bw, lng, lnb, qw, qb, kw, kb, vw, vb, sw, sb):
    return pl.pallas_call(
        _tc_mid_body,
        grid=(NBLK,),
        in_specs=[_rows(HID), _rows(HID), _rows(HID),
                  _full((3 * HID, 1)), _full((1, HID)), _full((1, HID)),
                  _full((HID, QK)), _full((1, QK)),
                  _full((HID, QK)), _full((1, QK)),
                  _full((HID, QK)), _full((1, QK)),
                  _full((HID, HID)), _full((1, HID))],
        out_specs=[_rows(HID), _rows(QKP), _rows(QKP), _rows(QKP),
                   _rows(HID)],
        out_shape=[_sds((NP, HID)), _sds((NP, QKP)), _sds((NP, QKP)),
                   _sds((NP, QKP)), _sds((NP, HID))],
    )(p0, p1, r, bw, lng, lnb, qw, qb, kw, kb, vw, vb, sw, sb)


def _tc_pool_body(p0, p1, r, h0, bw, lng, lnb, batch3, psum_o, pcnt_o):
    i = pl.program_id(0)
    o = (p0[...] + p1[...]) * (1.0 / H)
    hx = _gate(o, r[...], bw[...]) + h0[...]
    y = _ln_relu(hx, lng[...], lnb[...])
    b = batch3[0]
    seg = lax.broadcasted_iota(jnp.int32, (G, 1), 0)
    oh = (b == seg).astype(F32)

    @pl.when(i == 0)
    def _():
        psum_o[...] = jnp.zeros_like(psum_o)
        pcnt_o[...] = jnp.zeros_like(pcnt_o)

    psum_o[...] += jnp.dot(oh, y, preferred_element_type=F32)
    pcnt_o[...] += jnp.broadcast_to(jnp.sum(oh, axis=1, keepdims=True),
                                    (G, HID))


def _tc_pool(p0, p1, r, h0, bw, lng, lnb, batch3):
    return pl.pallas_call(
        _tc_pool_body,
        grid=(NBLK,),
        in_specs=[_rows(HID), _rows(HID), _rows(HID), _rows(HID),
                  _full((3 * HID, 1)), _full((1, HID)), _full((1, HID)),
                  pl.BlockSpec((1, 1, RB), lambda i: (i, 0, 0))],
        out_specs=[_full((G, HID)), _full((G, HID))],
        out_shape=[_sds((G, HID)), _sds((G, HID))],
    )(p0, p1, r, h0, bw, lng, lnb, batch3)


def _tc_cls_body(psum, pcnt, w1, b1, w2, b2, out_o):
    pooled = psum[...] / jnp.maximum(pcnt[...], 1.0)
    hc = jnp.maximum(jnp.dot(pooled, w1[...], preferred_element_type=F32)
                     + b1[...], 0.0)
    out_o[...] = jnp.dot(hc, w2[...], preferred_element_type=F32) + b2[...]


def _tc_cls(psum, pcnt, w1, b1, w2, b2):
    return pl.pallas_call(
        _tc_cls_body,
        grid=(1,),
        in_specs=[_full((G, HID)), _full((G, HID)),
                  _full((HID, HID // 2)), _full((1, HID // 2)),
                  _full((HID // 2, 1)), _full((1, 1))],
        out_specs=[_full((G, 1))],
        out_shape=[_sds((G, 1))],
    )(psum, pcnt, w1, b1, w2, b2)


def _tc_densum_body(d0, d1, o_o):
    o_o[...] = d0[...] + d1[...]


def _tc_densum(d0, d1):
    return pl.pallas_call(
        _tc_densum_body,
        grid=(NBLK,),
        in_specs=[_rows(HID), _rows(HID)],
        out_specs=[_rows(HID)],
        out_shape=[_sds((NP, HID))],
    )(d0, d1)[0]


def _perm_w(w):
    return w.reshape(-1, H, HID).transpose(0, 2, 1).reshape(-1, H * HID)


def _perm_b(b):
    return b.reshape(H, HID).T.reshape(1, -1)


def kernel(x, edge_index, batch, node_depth, child_index, emb_node, emb_depth,
           emb_child, proj_w, proj_b, l0_qw, l0_qb, l0_kw, l0_kb, l0_vw,
           l0_vb, l0_sw, l0_sb, l0_bw, l0_ln_g, l0_ln_b, l1_qw, l1_qb, l1_kw,
           l1_kb, l1_vw, l1_vb, l1_sw, l1_sb, l1_bw, l1_ln_g, l1_ln_b,
           cls_w1, cls_b1, cls_w2, cls_b2):
    pad = NP - N
    xi = jnp.pad(x.astype(jnp.int32), (0, pad))
    dep3 = jnp.pad(node_depth.astype(jnp.int32), (0, pad)).reshape(
        NBLK, 1, RB)
    chi3 = jnp.pad(child_index.astype(jnp.int32), (0, pad)).reshape(
        NBLK, 1, RB)
    batch3 = jnp.pad(batch.astype(jnp.int32), (0, pad),
                     constant_values=G).reshape(NBLK, 1, RB)
    srcf = edge_index[0].astype(jnp.int32)
    dstf = edge_index[1].astype(jnp.int32)
    src = srcf.reshape(NW, NREFILL, IDXB, CH)
    dst = dstf.reshape(NW, NREFILL, IDXB, CH)
    srcg = srcf.reshape(NW, AGG_NREFILL, AGG_IDXB, CH)
    dstg = dstf.reshape(NW, AGG_NREFILL, AGG_IDXB, CH)

    nfa = _embed(xi, emb_node)

    P = jnp.array(PERM128, dtype=jnp.int32)

    def permc(w):
        return w[:, P]

    def permr(w):
        return w[P]

    def perm_bw(bw):
        return jnp.concatenate([bw[0:HID][P], bw[HID:2 * HID][P],
                                bw[2 * HID:][P]], axis=0)

    def eo(w):
        return jnp.concatenate([w[:, 0::2], w[:, 1::2]], axis=1)

    def eo_b(b2):
        return jnp.concatenate([b2[:, 0::2], b2[:, 1::2]], axis=1)

    h, qp0, kp0, v0, r0 = _tc_proj0(
        nfa, dep3, chi3, emb_depth, emb_child,
        proj_w, proj_b.reshape(1, -1),
        eo(_perm_w(l0_qw)), eo_b(_perm_b(l0_qb)),
        eo(_perm_w(l0_kw)), eo_b(_perm_b(l0_kb)),
        eo(l0_vw), eo_b(l0_vb.reshape(1, -1)), permc(l0_sw),
        l0_sb[P].reshape(1, -1))

    ex0, den0 = _attn_ex(qp0, kp0, src, dst)
    outp0 = _attn_agg(v0, ex0, _tc_densum(den0[0], den0[1]), srcg, dstg)

    h0, qp1, kp1, v1, r1 = _tc_mid(
        outp0[0], outp0[1], r0, perm_bw(l0_bw),
        l0_ln_g[P].reshape(1, -1), l0_ln_b[P].reshape(1, -1),
        eo(_perm_w(permr(l1_qw))), eo_b(_perm_b(l1_qb)),
        eo(_perm_w(permr(l1_kw))), eo_b(_perm_b(l1_kb)),
        eo(permr(l1_vw)), eo_b(l1_vb.reshape(1, -1)),
        permc(permr(l1_sw)), l1_sb[P].reshape(1, -1))

    ex1, den1 = _attn_ex(qp1, kp1, src, dst)
    outp1 = _attn_agg(v1, ex1, _tc_densum(den1[0], den1[1]), srcg, dstg)

    psum, pcnt = _tc_pool(outp1[0], outp1[1], r1, h0, perm_bw(l1_bw),
                          l1_ln_g[P].reshape(1, -1),
                          l1_ln_b[P].reshape(1, -1), batch3)
    return _tc_cls(psum, pcnt, permr(cls_w1), cls_b1.reshape(1, -1),
                   cls_w2, cls_b2.reshape(1, -1))[0]


# speedup vs baseline: 22.8826x; 1.0640x over previous
import functools
import math

import jax
import jax.numpy as jnp
from jax import lax
from jax.experimental import pallas as pl
from jax.experimental.pallas import tpu as pltpu
from jax.experimental.pallas import tpu_sc as plsc

N = 10000
NP = 10240
E = 320000
EMB = 256
DD = 32
CD = 32
HID = 128
H = 8
G = 64
QK = H * HID
QKP = QK // 2

NC = 2
NS = 16
NW = NC * NS
EW = E // NW
CH = 16
NCHUNK = EW // CH
IDXB = 25
NREFILL = NCHUNK // IDXB
AGG_IDXB = 25
AGG_NREFILL = NCHUNK // AGG_IDXB
ROWS_T = NP // NS
NROW_W = NP // NW

LANES = [0, 4, 1, 5, 2, 6, 3, 7]

PERM128 = tuple([32 * (p // 16) + 2 * (p % 16) for p in range(64)]
                + [32 * (p // 16) + 2 * (p % 16) + 1 for p in range(64)])

_SC_MESH = dict(core_axis_name="c", subcore_axis_name="s",
                num_cores=NC, num_subcores=NS)

F32 = jnp.float32
BF16 = jnp.bfloat16


def _sds(shape, dtype=F32):
    return jax.ShapeDtypeStruct(shape, dtype)


@functools.partial(
    pl.kernel,
    out_type=_sds((NP, EMB)),
    mesh=plsc.VectorSubcoreMesh(**_SC_MESH),
    scratch_types=[
        pltpu.VMEM((64,), jnp.int32),
        pltpu.VMEM((64, EMB), F32),
        pltpu.SemaphoreType.DMA,
    ],
)
def _embed(x_h, tn_h, nfa_h, ib, nb, sem):
    cid = lax.axis_index("c")
    sid = lax.axis_index("s")
    wid = sid * NC + cid
    base0 = wid * NROW_W

    def chunk(i, _):
        base = base0 + i * 64
        pltpu.sync_copy(x_h.at[pl.ds(base, 64)], ib)
        pltpu.async_copy(tn_h.at[ib], nb, sem).wait()
        pltpu.sync_copy(nb, nfa_h.at[pl.ds(base, 64)])
        return _

    lax.fori_loop(0, NROW_W // 64, chunk, None)


@functools.partial(
    pl.kernel,
    out_type=(_sds((E, 16)), _sds((NC, NP, HID))),
    mesh=plsc.VectorSubcoreMesh(**_SC_MESH),
    scratch_types=[
        pltpu.VMEM((IDXB, CH), jnp.int32),
        pltpu.VMEM((IDXB, CH), jnp.int32),
        pltpu.VMEM((CH, QKP), F32),
        pltpu.VMEM((CH, QKP), F32),
        pltpu.VMEM((CH, QKP), F32),
        pltpu.VMEM((CH, QKP), F32),
        pltpu.VMEM((CH, 16), F32),
        pltpu.VMEM((CH, HID), F32),
        pltpu.VMEM((CH, HID), F32),
        pltpu.VMEM((8, HID), F32),
        pltpu.VMEM((32,), F32),
        pltpu.VMEM_SHARED((NP, HID), F32),
        pltpu.SemaphoreType.DMA,
        pltpu.SemaphoreType.DMA,
        pltpu.SemaphoreType.DMA,
        pltpu.SemaphoreType.DMA,
        pltpu.SemaphoreType.DMA,
        pltpu.SemaphoreType.DMA,
    ],
)
def _attn_ex(qp_h, kp_h, srcr_h, dstr_h, ex_h, den_h,
             src_i, dst_i, qba, kba, qbb, kbb, exb, exda, exdb, zb, tmp,
             den_sh, sqa, ska, sqb, skb, ssa, ssb):
    cid = lax.axis_index("c")
    sid = lax.axis_index("s")
    wid = sid * NC + cid
    zv = jnp.zeros((16,), F32)

    def zrow(i, _):
        for t in range(HID // 16):
            zb[i, pl.ds(16 * t, 16)] = zv
        return _

    lax.fori_loop(0, 8, zrow, None)

    def zex(e, _):
        for t in range(HID // 16):
            exda[e, pl.ds(16 * t, 16)] = zv
            exdb[e, pl.ds(16 * t, 16)] = zv
        return _

    lax.fori_loop(0, CH, zex, None)

    def zcopy(t, _):
        pltpu.sync_copy(zb, den_sh.at[pl.ds(sid * ROWS_T + t * 8, 8)])
        return _

    lax.fori_loop(0, ROWS_T // 8, zcopy, None)
    plsc.subcore_barrier()

    def prime():
        pltpu.async_copy(exda, den_h.at[cid, pl.ds(sid * ROWS_T, CH)], ssa)
        pltpu.async_copy(exdb,
                         den_h.at[cid, pl.ds(sid * ROWS_T + CH, CH)], ssb)

    def drain():
        pltpu.make_async_copy(exda, den_sh.at[dst_i.at[0]], ssa).wait()
        pltpu.make_async_copy(exdb, den_sh.at[dst_i.at[0]], ssb).wait()

    prime()
    scale = 1.0 / math.sqrt(float(HID))
    ebase = wid * EW
    mask_hi = jnp.int32(-65536)

    def split(v16):
        wi = lax.bitcast_convert_type(v16, jnp.int32)
        ev = lax.bitcast_convert_type(lax.shift_left(wi, 16), F32)
        od = lax.bitcast_convert_type(lax.bitwise_and(wi, mask_hi), F32)
        return ev, od

    def start(j, qb, kb, sq, sk):
        pltpu.async_copy(qp_h.at[dst_i.at[j]], qb, sq)
        pltpu.async_copy(kp_h.at[src_i.at[j]], kb, sk)

    def wait(qb, kb, sq, sk):
        pltpu.make_async_copy(qp_h.at[dst_i.at[0]], qb, sq).wait()
        pltpu.make_async_copy(kp_h.at[src_i.at[0]], kb, sk).wait()

    def compute(rj, j, qb, kb, exd, ss):
        pltpu.make_async_copy(exd, den_sh.at[dst_i.at[0]], ss).wait()

        def edge(e, _):
            acc_e = None
            acc_o = None
            for t in range(QKP // 16):
                qe, qo = split(qb[e, pl.ds(16 * t, 16)])
                ke, ko = split(kb[e, pl.ds(16 * t, 16)])
                if acc_e is None:
                    acc_e = qe * ke
                    acc_o = qo * ko
                else:
                    acc_e = acc_e + qe * ke
                    acc_o = acc_o + qo * ko
            folded = []
            for a in (acc_e, acc_o):
                tmp[pl.ds(0, 16)] = a
                tmp[pl.ds(16, 16)] = a
                s1 = a + tmp[pl.ds(8, 16)]
                tmp[pl.ds(0, 16)] = s1
                tmp[pl.ds(16, 16)] = s1
                folded.append(s1 + tmp[pl.ds(4, 16)])
            tmp[pl.ds(0, 16)] = folded[0]
            tmp[pl.ds(4, 16)] = folded[1]
            ev = jnp.exp(tmp[pl.ds(0, 16)] * scale)
            exb[e, :] = ev
            exd[e, pl.ds(0, 16)] = ev
            return _

        lax.fori_loop(0, CH, edge, None)
        pltpu.sync_copy(exb,
                        ex_h.at[pl.ds(ebase + (rj * IDXB + j) * CH, CH)])
        pltpu.async_copy(exd, den_sh.at[dst_i.at[j]], ss, add=True)

    def refill(rj, _):
        drain()
        pltpu.sync_copy(srcr_h.at[wid, rj], src_i)
        pltpu.sync_copy(dstr_h.at[wid, rj], dst_i)
        prime()
        start(0, qba, kba, sqa, ska)

        def pair(jj, _):
            j0 = 2 * jj
            start(j0 + 1, qbb, kbb, sqb, skb)
            wait(qba, kba, sqa, ska)
            compute(rj, j0, qba, kba, exda, ssa)
            start(j0 + 2, qba, kba, sqa, ska)
            wait(qbb, kbb, sqb, skb)
            compute(rj, j0 + 1, qbb, kbb, exdb, ssb)
            return _

        lax.fori_loop(0, (IDXB - 1) // 2, pair, None)
        wait(qba, kba, sqa, ska)
        compute(rj, IDXB - 1, qba, kba, exda, ssa)
        return _

    lax.fori_loop(0, NREFILL, refill, None)
    drain()
    plsc.subcore_barrier()

    def dump(t, _):
        off = sid * ROWS_T + t * 64
        pltpu.sync_copy(den_sh.at[pl.ds(off, 64)],
                        den_h.at[cid, pl.ds(off, 64)])
        return _

    lax.fori_loop(0, ROWS_T // 64, dump, None)


@functools.partial(
    pl.kernel,
    out_type=_sds((NC, NP, HID)),
    mesh=plsc.VectorSubcoreMesh(**_SC_MESH),
    scratch_types=[
        pltpu.VMEM((AGG_IDXB, CH), jnp.int32),
        pltpu.VMEM((AGG_IDXB, CH), jnp.int32),
        pltpu.VMEM((CH, QKP), F32),
        pltpu.VMEM((CH, QKP), F32),
        pltpu.VMEM((CH, 16), F32),
        pltpu.VMEM((CH, 16), F32),
        pltpu.VMEM((CH, HID), F32),
        pltpu.VMEM((CH, HID), F32),
        pltpu.VMEM((CH, HID), F32),
        pltpu.VMEM((CH, HID), F32),
        pltpu.VMEM((8, HID), F32),
        pltpu.VMEM_SHARED((NP, HID), F32),
        pltpu.SemaphoreType.DMA,
        pltpu.SemaphoreType.DMA,
        pltpu.SemaphoreType.DMA,
        pltpu.SemaphoreType.DMA,
        pltpu.SemaphoreType.DMA,
        pltpu.SemaphoreType.DMA,
        pltpu.SemaphoreType.DMA,
        pltpu.SemaphoreType.DMA,
    ],
)
def _attn_agg(v_h, ex_h, den_h, srcr_h, dstr_h, outp_h,
              src_i, dst_i, vba, vbb, exba, exbb, dba, dbb, mba, mbb, zb,
              out_sh, sva, svb, sda, sdb, sea, seb, ssa, ssb):
    cid = lax.axis_index("c")
    sid = lax.axis_index("s")
    wid = sid * NC + cid
    zv = jnp.zeros((16,), F32)
    mask_hi = jnp.int32(-65536)

    def split(v16):
        wi = lax.bitcast_convert_type(v16, jnp.int32)
        ev = lax.bitcast_convert_type(lax.shift_left(wi, 16), F32)
        od = lax.bitcast_convert_type(lax.bitwise_and(wi, mask_hi), F32)
        return ev, od

    def zrow(i, _):
        for t in range(HID // 16):
            zb[i, pl.ds(16 * t, 16)] = zv
        return _

    lax.fori_loop(0, 8, zrow, None)

    def zcopy(t, _):
        pltpu.sync_copy(zb, out_sh.at[pl.ds(sid * ROWS_T + t * 8, 8)])
        return _

    lax.fori_loop(0, ROWS_T // 8, zcopy, None)
    plsc.subcore_barrier()

    def prime():
        pltpu.async_copy(mba, outp_h.at[cid, pl.ds(sid * ROWS_T, CH)], ssa)
        pltpu.async_copy(mbb,
                         outp_h.at[cid, pl.ds(sid * ROWS_T + CH, CH)], ssb)

    def drain():
        pltpu.make_async_copy(mba, out_sh.at[dst_i.at[0]], ssa).wait()
        pltpu.make_async_copy(mbb, out_sh.at[dst_i.at[0]], ssb).wait()

    prime()
    ebase = wid * EW

    def start(rj, j, vb, db, eb, sv, sd, se):
        pltpu.async_copy(v_h.at[src_i.at[j]], vb, sv)
        pltpu.async_copy(den_h.at[dst_i.at[j]], db, sd)
        pltpu.async_copy(
            ex_h.at[pl.ds(ebase + (rj * AGG_IDXB + j) * CH, CH)], eb, se)

    def wait(vb, db, eb, sv, sd, se):
        pltpu.make_async_copy(v_h.at[src_i.at[0]], vb, sv).wait()
        pltpu.make_async_copy(den_h.at[dst_i.at[0]], db, sd).wait()
        pltpu.make_async_copy(ex_h.at[pl.ds(0, CH)], eb, se).wait()

    def compute(j, vb, db, eb, mb, ss):
        pltpu.make_async_copy(mb, out_sh.at[dst_i.at[0]], ss).wait()

        def edge(e, _):
            denv = db[e, pl.ds(0, 16)]
            alpha = eb[e, :] / jnp.maximum(denv, 1e-16)
            msgs_e = None
            msgs_o = None
            for hh in range(H):
                ah = alpha[LANES[hh]]
                hb = hh * (HID // 2)
                if msgs_e is None:
                    pairs = [split(vb[e, pl.ds(hb + 16 * u, 16)])
                             for u in range(HID // 32)]
                    msgs_e = [ah * pe for pe, _po in pairs]
                    msgs_o = [ah * po for _pe, po in pairs]
                else:
                    for u in range(HID // 32):
                        ve, vo = split(vb[e, pl.ds(hb + 16 * u, 16)])
                        msgs_e[u] = msgs_e[u] + ah * ve
                        msgs_o[u] = msgs_o[u] + ah * vo
            for u in range(HID // 32):
                mb[e, pl.ds(16 * u, 16)] = msgs_e[u]
                mb[e, pl.ds(64 + 16 * u, 16)] = msgs_o[u]
            return _

        lax.fori_loop(0, CH, edge, None)
        pltpu.async_copy(mb, out_sh.at[dst_i.at[j]], ss, add=True)

    def refill(rj, _):
        drain()
        pltpu.sync_copy(srcr_h.at[wid, rj], src_i)
        pltpu.sync_copy(dstr_h.at[wid, rj], dst_i)
        prime()
        start(rj, 0, vba, dba, exba, sva, sda, sea)

        def pair(jj, _):
            j0 = 2 * jj
            start(rj, j0 + 1, vbb, dbb, exbb, svb, sdb, seb)
            wait(vba, dba, exba, sva, sda, sea)
            compute(j0, vba, dba, exba, mba, ssa)
            start(rj, j0 + 2, vba, dba, exba, sva, sda, sea)
            wait(vbb, dbb, exbb, svb, sdb, seb)
            compute(j0 + 1, vbb, dbb, exbb, mbb, ssb)
            return _

        lax.fori_loop(0, (AGG_IDXB - 1) // 2, pair, None)
        wait(vba, dba, exba, sva, sda, sea)
        compute(AGG_IDXB - 1, vba, dba, exba, mba, ssa)
        return _

    lax.fori_loop(0, AGG_NREFILL, refill, None)
    drain()
    plsc.subcore_barrier()

    def dump(t, _):
        off = sid * ROWS_T + t * 64
        pltpu.sync_copy(out_sh.at[pl.ds(off, 64)],
                        outp_h.at[cid, pl.ds(off, 64)])
        return _

    lax.fori_loop(0, ROWS_T // 64, dump, None)


RB = 512
NBLK = NP // RB

DDEP = 51
DCHI = 21


def _pack2(a, b):
    ai = lax.bitcast_convert_type(a, jnp.int32)
    bi = lax.bitcast_convert_type(b, jnp.int32)
    ar = ai + 32767 + lax.bitwise_and(lax.shift_right_logical(ai, 16), 1)
    br = bi + 32767 + lax.bitwise_and(lax.shift_right_logical(bi, 16), 1)
    lo = lax.shift_right_logical(ar, 16)
    hi = lax.bitwise_and(br, jnp.int32(-65536))
    return lax.bitcast_convert_type(lax.bitwise_or(lo, hi), F32)


def _onehot_lookup(ids3, table, n_ids):
    ids = ids3[0]
    oh = (lax.broadcasted_iota(jnp.int32, (n_ids, 1), 0) == ids).astype(F32)
    return lax.dot_general(oh, table, (((0,), (0,)), ((), ())),
                           preferred_element_type=F32)


def _tc_proj0_body(nfa, dep3, chi3, td, tc, pw, pb, qw, qb, kw, kb, vw, vb,
                   sw, sb, h_o, qp_o, kp_o, v_o, r_o):
    pwv = pw[...]
    nfb = _onehot_lookup(dep3, td[...], DDEP)
    nfc = _onehot_lookup(chi3, tc[...], DCHI)
    h = (jnp.dot(nfa[...], pwv[0:EMB], preferred_element_type=F32)
         + jnp.dot(nfb, pwv[EMB:EMB + DD], preferred_element_type=F32)
         + jnp.dot(nfc, pwv[EMB + DD:], preferred_element_type=F32)
         + pb[...])
    h = jnp.maximum(h, 0.0)
    h_o[...] = h
    qc = jnp.dot(h, qw[...], preferred_element_type=F32) + qb[...]
    qp_o[...] = _pack2(qc[:, 0:QKP], qc[:, QKP:])
    kc = jnp.dot(h, kw[...], preferred_element_type=F32) + kb[...]
    kp_o[...] = _pack2(kc[:, 0:QKP], kc[:, QKP:])
    vc = jnp.dot(h, vw[...], preferred_element_type=F32) + vb[...]
    v_o[...] = _pack2(vc[:, 0:QKP], vc[:, QKP:])
    r_o[...] = jnp.dot(h, sw[...], preferred_element_type=F32) + sb[...]


def _full(shape):
    nd = len(shape)
    return pl.BlockSpec(shape, lambda i: (0,) * nd)


def _rows(cols):
    return pl.BlockSpec((RB, cols), lambda i: (i, 0))


def _tc_proj0(nfa, dep3, chi3, td, tc, pw, pb, qw, qb, kw, kb, vw, vb,
              sw, sb):
    return pl.pallas_call(
        _tc_proj0_body,
        grid=(NBLK,),
        in_specs=[_rows(EMB),
                  pl.BlockSpec((1, 1, RB), lambda i: (i, 0, 0)),
                  pl.BlockSpec((1, 1, RB), lambda i: (i, 0, 0)),
                  _full((DDEP, DD)), _full((DCHI, CD)),
                  _full((EMB + DD + CD, EMB)), _full((1, EMB)),
                  _full((EMB, QK)), _full((1, QK)),
                  _full((EMB, QK)), _full((1, QK)),
                  _full((EMB, QK)), _full((1, QK)),
                  _full((EMB, HID)), _full((1, HID))],
        out_specs=[_rows(EMB), _rows(QKP), _rows(QKP), _rows(QKP),
                   _rows(HID)],
        out_shape=[_sds((NP, EMB)), _sds((NP, QKP)), _sds((NP, QKP)),
                   _sds((NP, QKP)), _sds((NP, HID))],
    )(nfa, dep3, chi3, td, tc, pw, pb, qw, qb, kw, kb, vw, vb, sw, sb)


def _gate(o, rr, bwv):
    z = (jnp.dot(o, bwv[0:HID], preferred_element_type=F32)
         + jnp.dot(rr, bwv[HID:2 * HID], preferred_element_type=F32)
         + jnp.dot(o - rr, bwv[2 * HID:], preferred_element_type=F32))
    g = jax.nn.sigmoid(z)
    return g * rr + (1.0 - g) * o


def _ln_relu(hx, lng, lnb):
    mu = jnp.mean(hx, axis=1, keepdims=True)
    var = jnp.mean((hx - mu) * (hx - mu), axis=1, keepdims=True)
    y = (hx - mu) / jnp.sqrt(var + 1e-5) * lng + lnb
    return jnp.maximum(y, 0.0)


def _tc_mid_body(p0, p1, r, bw, lng, lnb, qw, qb, kw, kb, vw, vb, sw, sb,
                 h0_o, qp_o, kp_o, v_o, r_o):
    o = (p0[...] + p1[...]) * (1.0 / H)
    hx = _gate(o, r[...], bw[...])
    y = _ln_relu(hx, lng[...], lnb[...])
    h0_o[...] = y
    qc = jnp.dot(y, qw[...], preferred_element_type=F32) + qb[...]
    qp_o[...] = _pack2(qc[:, 0:QKP], qc[:, QKP:])
    kc = jnp.dot(y, kw[...], preferred_element_type=F32) + kb[...]
    kp_o[...] = _pack2(kc[:, 0:QKP], kc[:, QKP:])
    vc = jnp.dot(y, vw[...], preferred_element_type=F32) + vb[...]
    v_o[...] = _pack2(vc[:, 0:QKP], vc[:, QKP:])
    r_o[...] = jnp.dot(y, sw[...], preferred_element_type=F32) + sb[...]


def _tc_mid(p0, p1, r, bw, lng, lnb, qw, qb, kw, kb, vw, vb, sw, sb):
    return pl.pallas_call(
        _tc_mid_body,
        grid=(NBLK,),
        in_specs=[_rows(HID), _rows(HID), _rows(HID),
                  _full((3 * HID, 1)), _full((1, HID)), _full((1, HID)),
                  _full((HID, QK)), _full((1, QK)),
                  _full((HID, QK)), _full((1, QK)),
                  _full((HID, QK)), _full((1, QK)),
                  _full((HID, HID)), _full((1, HID))],
        out_specs=[_rows(HID), _rows(QKP), _rows(QKP), _rows(QKP),
                   _rows(HID)],
        out_shape=[_sds((NP, HID)), _sds((NP, QKP)), _sds((NP, QKP)),
                   _sds((NP, QKP)), _sds((NP, HID))],
    )(p0, p1, r, bw, lng, lnb, qw, qb, kw, kb, vw, vb, sw, sb)


def _tc_pool_body(p0, p1, r, h0, bw, lng, lnb, batch3, psum_o, pcnt_o):
    i = pl.program_id(0)
    o = (p0[...] + p1[...]) * (1.0 / H)
    hx = _gate(o, r[...], bw[...]) + h0[...]
    y = _ln_relu(hx, lng[...], lnb[...])
    b = batch3[0]
    seg = lax.broadcasted_iota(jnp.int32, (G, 1), 0)
    oh = (b == seg).astype(F32)

    @pl.when(i == 0)
    def _():
        psum_o[...] = jnp.zeros_like(psum_o)
        pcnt_o[...] = jnp.zeros_like(pcnt_o)

    psum_o[...] += jnp.dot(oh, y, preferred_element_type=F32)
    pcnt_o[...] += jnp.broadcast_to(jnp.sum(oh, axis=1, keepdims=True),
                                    (G, HID))


def _tc_pool(p0, p1, r, h0, bw, lng, lnb, batch3):
    return pl.pallas_call(
        _tc_pool_body,
        grid=(NBLK,),
        in_specs=[_rows(HID), _rows(HID), _rows(HID), _rows(HID),
                  _full((3 * HID, 1)), _full((1, HID)), _full((1, HID)),
                  pl.BlockSpec((1, 1, RB), lambda i: (i, 0, 0))],
        out_specs=[_full((G, HID)), _full((G, HID))],
        out_shape=[_sds((G, HID)), _sds((G, HID))],
    )(p0, p1, r, h0, bw, lng, lnb, batch3)


def _tc_cls_body(psum, pcnt, w1, b1, w2, b2, out_o):
    pooled = psum[...] / jnp.maximum(pcnt[...], 1.0)
    hc = jnp.maximum(jnp.dot(pooled, w1[...], preferred_element_type=F32)
                     + b1[...], 0.0)
    out_o[...] = jnp.dot(hc, w2[...], preferred_element_type=F32) + b2[...]


def _tc_cls(psum, pcnt, w1, b1, w2, b2):
    return pl.pallas_call(
        _tc_cls_body,
        grid=(1,),
        in_specs=[_full((G, HID)), _full((G, HID)),
                  _full((HID, HID // 2)), _full((1, HID // 2)),
                  _full((HID // 2, 1)), _full((1, 1))],
        out_specs=[_full((G, 1))],
        out_shape=[_sds((G, 1))],
    )(psum, pcnt, w1, b1, w2, b2)


def _tc_densum_body(d0, d1, o_o):
    o_o[...] = d0[...] + d1[...]


def _tc_densum(d0, d1):
    return pl.pallas_call(
        _tc_densum_body,
        grid=(NBLK,),
        in_specs=[_rows(HID), _rows(HID)],
        out_specs=[_rows(HID)],
        out_shape=[_sds((NP, HID))],
    )(d0, d1)[0]


def _perm_w(w):
    return w.reshape(-1, H, HID).transpose(0, 2, 1).reshape(-1, H * HID)


def _perm_b(b):
    return b.reshape(H, HID).T.reshape(1, -1)


def kernel(x, edge_index, batch, node_depth, child_index, emb_node, emb_depth,
           emb_child, proj_w, proj_b, l0_qw, l0_qb, l0_kw, l0_kb, l0_vw,
           l0_vb, l0_sw, l0_sb, l0_bw, l0_ln_g, l0_ln_b, l1_qw, l1_qb, l1_kw,
           l1_kb, l1_vw, l1_vb, l1_sw, l1_sb, l1_bw, l1_ln_g, l1_ln_b,
           cls_w1, cls_b1, cls_w2, cls_b2):
    pad = NP - N
    xi = jnp.pad(x.astype(jnp.int32), (0, pad))
    dep3 = jnp.pad(node_depth.astype(jnp.int32), (0, pad)).reshape(
        NBLK, 1, RB)
    chi3 = jnp.pad(child_index.astype(jnp.int32), (0, pad)).reshape(
        NBLK, 1, RB)
    batch3 = jnp.pad(batch.astype(jnp.int32), (0, pad),
                     constant_values=G).reshape(NBLK, 1, RB)
    srcf = edge_index[0].astype(jnp.int32)
    dstf = edge_index[1].astype(jnp.int32)
    src = srcf.reshape(NW, NREFILL, IDXB, CH)
    dst = dstf.reshape(NW, NREFILL, IDXB, CH)
    srcg = srcf.reshape(NW, AGG_NREFILL, AGG_IDXB, CH)
    dstg = dstf.reshape(NW, AGG_NREFILL, AGG_IDXB, CH)

    nfa = _embed(xi, emb_node)

    P = jnp.array(PERM128, dtype=jnp.int32)

    def permc(w):
        return w[:, P]

    def permr(w):
        return w[P]

    def perm_bw(bw):
        return jnp.concatenate([bw[0:HID][P], bw[HID:2 * HID][P],
                                bw[2 * HID:][P]], axis=0)

    def eo(w):
        return jnp.concatenate([w[:, 0::2], w[:, 1::2]], axis=1)

    def eo_b(b2):
        return jnp.concatenate([b2[:, 0::2], b2[:, 1::2]], axis=1)

    h, qp0, kp0, v0, r0 = _tc_proj0(
        nfa, dep3, chi3, emb_depth, emb_child,
        proj_w, proj_b.reshape(1, -1),
        eo(_perm_w(l0_qw)), eo_b(_perm_b(l0_qb)),
        eo(_perm_w(l0_kw)), eo_b(_perm_b(l0_kb)),
        eo(l0_vw), eo_b(l0_vb.reshape(1, -1)), permc(l0_sw),
        l0_sb[P].reshape(1, -1))

    ex0, den0 = _attn_ex(qp0, kp0, src, dst)
    outp0 = _attn_agg(v0, ex0, _tc_densum(den0[0], den0[1]), srcg, dstg)

    h0, qp1, kp1, v1, r1 = _tc_mid(
        outp0[0], outp0[1], r0, perm_bw(l0_bw),
        l0_ln_g[P].reshape(1, -1), l0_ln_b[P].reshape(1, -1),
        eo(_perm_w(permr(l1_qw))), eo_b(_perm_b(l1_qb)),
        eo(_perm_w(permr(l1_kw))), eo_b(_perm_b(l1_kb)),
        eo(permr(l1_vw)), eo_b(l1_vb.reshape(1, -1)),
        permc(permr(l1_sw)), l1_sb[P].reshape(1, -1))

    ex1, den1 = _attn_ex(qp1, kp1, src, dst)
    outp1 = _attn_agg(v1, ex1, _tc_densum(den1[0], den1[1]), srcg, dstg)

    psum, pcnt = _tc_pool(outp1[0], outp1[1], r1, h0, perm_bw(l1_bw),
                          l1_ln_g[P].reshape(1, -1),
                          l1_ln_b[P].reshape(1, -1), batch3)
    return _tc_cls(psum, pcnt, permr(cls_w1), cls_b1.reshape(1, -1),
                   cls_w2, cls_b2.reshape(1, -1))[0]
